# Initial kernel scaffold; baseline (speedup 1.0000x reference)
import jax
import jax.numpy as jnp
from jax.experimental import pallas as pl


def kernel(edge_index, edge_values, const_values, objective_multipliers, integer_mask, cu_W1, cu_b1, cu_W2, cu_b2, mq_W1, mq_b1, mq_W2, mq_b2, vu_W1, vu_b1, vu_W2, vu_b2, out_W1, out_b1, out_W2, out_b2):
    raise NotImplementedError("write your pallas kernel here")


# speedup vs baseline: 2.1042x; 2.1042x over previous
import functools
import jax
import jax.numpy as jnp
from jax import lax
from jax.experimental import pallas as pl
from jax.experimental.pallas import tpu as pltpu
from jax.experimental.pallas import tpu_sc as plsc

FM = 64
CHUNK = 256
SUB = 128
ROW_R = 2000

_GD = lax.GatherDimensionNumbers(offset_dims=(), collapsed_slice_dims=(0,),
                                 start_index_map=(0,))


def _bcast_lane(vv, j):
  idx = jnp.full((16, 1), j, jnp.int32)
  return lax.gather(vv, idx, _GD, (1,),
                    mode=lax.GatherScatterMode.PROMISE_IN_BOUNDS)


def _make_segsum(n_out, n_edges_pad, fm):
  half = n_out // 2
  wrow = 50
  acc_rows = ((half + 1 + wrow - 1) // wrow) * wrow
  junk = acc_rows - 1
  tile_edges = n_edges_pad // 16
  nchunk = tile_edges // CHUNK
  zchunks = acc_rows // wrow
  wchunks = half // wrow
  mesh = plsc.VectorSubcoreMesh(core_axis_name="c", subcore_axis_name="s")

  def body(gidx, sidx, vals, table, out,
           graw, draw, vraw, gidx2, sloc2, veff, rows, wstage, acc, sem):
    cid = lax.axis_index("c")
    sid = lax.axis_index("s")
    base = cid * half

    def zb(r, _):
      for k in range(fm // 16):
        wstage[r, pl.ds(k * 16, 16)] = jnp.zeros((16,), jnp.float32)
      return _
    lax.fori_loop(0, wrow, zb, None)
    for j in range((zchunks + 15) // 16):
      c = sid + 16 * j
      @pl.when(c < zchunks)
      def _():
        pltpu.sync_copy(wstage, acc.at[pl.ds(c * wrow, wrow), :])
    plsc.subcore_barrier()

    ebase = sid * tile_edges

    def chunk_body(ci, _):
      off = ebase + ci * CHUNK
      pltpu.sync_copy(gidx.at[pl.ds(off, CHUNK)], graw)
      pltpu.sync_copy(sidx.at[pl.ds(off, CHUNK)], draw)
      pltpu.sync_copy(vals.at[pl.ds(off, CHUNK)], vraw)

      def grp(g, _):
        gv = graw[pl.ds(g * 16, 16)]
        d = draw[pl.ds(g * 16, 16)]
        v = vraw[pl.ds(g * 16, 16)]
        inh = (d >= base) & (d < base + half)
        dl = jnp.where(inh, d - base, junk)
        ve = jnp.where(inh, v, 0.0)
        j = g // 8
        col = (g % 8) * 16
        gidx2[j, pl.ds(col, 16)] = gv
        sloc2[j, pl.ds(col, 16)] = dl
        veff[pl.ds(g * 16, 16)] = ve
        return _
      lax.fori_loop(0, CHUNK // 16, grp, None)

      hs = [pltpu.async_copy(table.at[gidx2.at[j]],
                             rows.at[pl.ds(j * SUB, SUB)], sem)
            for j in range(CHUNK // SUB)]
      for h in hs:
        h.wait()

      def scale(g, _):
        vv = veff[pl.ds(g * 16, 16)]
        for j16 in range(16):
          e = g * 16 + j16
          b = _bcast_lane(vv, j16)
          for k in range(fm // 16):
            r = rows[e, pl.ds(k * 16, 16)]
            rows[e, pl.ds(k * 16, 16)] = r * b
        return _
      lax.fori_loop(0, CHUNK // 16, scale, None)

      for j in range(CHUNK // SUB):
        pltpu.sync_copy(rows.at[pl.ds(j * SUB, SUB)],
                        acc.at[sloc2.at[j]], add=True)
      return _
    lax.fori_loop(0, nchunk, chunk_body, None)
    plsc.subcore_barrier()

    for j in range((wchunks + 15) // 16):
      c = sid + 16 * j
      @pl.when(c < wchunks)
      def _():
        pltpu.sync_copy(acc.at[pl.ds(c * wrow, wrow), :], wstage)
        pltpu.sync_copy(wstage, out.at[pl.ds(base + c * wrow, wrow), :])

  return pl.kernel(
      body,
      out_type=jax.ShapeDtypeStruct((n_out, fm), jnp.float32),
      mesh=mesh,
      compiler_params=pltpu.CompilerParams(use_tc_tiling_on_sc=False),
      scratch_types=[
          pltpu.VMEM((CHUNK,), jnp.int32),
          pltpu.VMEM((CHUNK,), jnp.int32),
          pltpu.VMEM((CHUNK,), jnp.float32),
          pltpu.VMEM((CHUNK // SUB, SUB), jnp.int32),
          pltpu.VMEM((CHUNK // SUB, SUB), jnp.int32),
          pltpu.VMEM((CHUNK,), jnp.float32),
          pltpu.VMEM((CHUNK, fm), jnp.float32),
          pltpu.VMEM((wrow, fm), jnp.float32),
          pltpu.VMEM_SHARED((acc_rows, fm), jnp.float32),
          pltpu.SemaphoreType.DMA,
      ],
  )


def _make_scalers(n_out, n_edges_pad):
  half = n_out // 2
  acc_n = ((half + 1 + 511) // 512) * 512
  junk = acc_n - 1
  tile_edges = n_edges_pad // 16
  nchunk = tile_edges // CHUNK
  zchunks = acc_n // 512
  wchunks = half // 1000
  mesh = plsc.VectorSubcoreMesh(core_axis_name="c", subcore_axis_name="s")

  def body(src, dst, vals, cs_out, vs_out,
           sraw, draw, vraw, sloc2, dloc2, veffs, veffd, zbuf, wstage,
           accc, accv, sem):
    cid = lax.axis_index("c")
    sid = lax.axis_index("s")
    base = cid * half

    def zb(r, _):
      zbuf[pl.ds(r * 16, 16)] = jnp.zeros((16,), jnp.float32)
      return _
    lax.fori_loop(0, 32, zb, None)
    for j in range((zchunks + 15) // 16):
      c = sid + 16 * j
      @pl.when(c < zchunks)
      def _():
        pltpu.sync_copy(zbuf, accc.at[pl.ds(c * 512, 512)])
        pltpu.sync_copy(zbuf, accv.at[pl.ds(c * 512, 512)])
    plsc.subcore_barrier()

    ebase = sid * tile_edges

    def chunk_body(ci, _):
      off = ebase + ci * CHUNK
      pltpu.sync_copy(src.at[pl.ds(off, CHUNK)], sraw)
      pltpu.sync_copy(dst.at[pl.ds(off, CHUNK)], draw)
      pltpu.sync_copy(vals.at[pl.ds(off, CHUNK)], vraw)

      def grp(g, _):
        s = sraw[pl.ds(g * 16, 16)]
        d = draw[pl.ds(g * 16, 16)]
        v = vraw[pl.ds(g * 16, 16)]
        inh_d = (d >= base) & (d < base + half)
        inh_s = (s >= base) & (s < base + half)
        j = g // 8
        col = (g % 8) * 16
        dloc2[j, pl.ds(col, 16)] = jnp.where(inh_d, d - base, junk)
        sloc2[j, pl.ds(col, 16)] = jnp.where(inh_s, s - base, junk)
        veffd[pl.ds(g * 16, 16)] = jnp.where(inh_d, v, 0.0)
        veffs[pl.ds(g * 16, 16)] = jnp.where(inh_s, v, 0.0)
        return _
      lax.fori_loop(0, CHUNK // 16, grp, None)

      for j in range(CHUNK // SUB):
        pltpu.sync_copy(veffd.at[pl.ds(j * SUB, SUB)],
                        accc.at[dloc2.at[j]], add=True)
        pltpu.sync_copy(veffs.at[pl.ds(j * SUB, SUB)],
                        accv.at[sloc2.at[j]], add=True)
      return _
    lax.fori_loop(0, nchunk, chunk_body, None)
    plsc.subcore_barrier()

    for j in range((wchunks + 15) // 16):
      c = sid + 16 * j
      @pl.when(c < wchunks)
      def _():
        pltpu.sync_copy(accc.at[pl.ds(c * 1000, 1000)], wstage)
        pltpu.sync_copy(wstage, cs_out.at[pl.ds(base + c * 1000, 1000)])
        pltpu.sync_copy(accv.at[pl.ds(c * 1000, 1000)], wstage)
        pltpu.sync_copy(wstage, vs_out.at[pl.ds(base + c * 1000, 1000)])

  return pl.kernel(
      body,
      out_type=(jax.ShapeDtypeStruct((n_out,), jnp.float32),
                jax.ShapeDtypeStruct((n_out,), jnp.float32)),
      mesh=mesh,
      compiler_params=pltpu.CompilerParams(use_tc_tiling_on_sc=False),
      scratch_types=[
          pltpu.VMEM((CHUNK,), jnp.int32),
          pltpu.VMEM((CHUNK,), jnp.int32),
          pltpu.VMEM((CHUNK,), jnp.float32),
          pltpu.VMEM((CHUNK // SUB, SUB), jnp.int32),
          pltpu.VMEM((CHUNK // SUB, SUB), jnp.int32),
          pltpu.VMEM((CHUNK,), jnp.float32),
          pltpu.VMEM((CHUNK,), jnp.float32),
          pltpu.VMEM((512,), jnp.float32),
          pltpu.VMEM((1000,), jnp.float32),
          pltpu.VMEM_SHARED((acc_n,), jnp.float32),
          pltpu.VMEM_SHARED((acc_n,), jnp.float32),
          pltpu.SemaphoreType.DMA,
      ],
  )


def _norm(x):
  m = jnp.mean(x, axis=-1, keepdims=True)
  v = jnp.sum((x - m) * (x - m), axis=-1, keepdims=True) * (1.0 / (FM - 1))
  return x / (jnp.sqrt(v) + 1e-5)


def _dot(a, b):
  return jnp.dot(a, b, preferred_element_type=jnp.float32)


def _stage1_body(v_ref, om_ref, w1_ref, b1_ref, w2_ref, b2_ref,
                 query_ref, objl_ref):
  x = jax.nn.relu(_dot(v_ref[...], w1_ref[...]) + b1_ref[...])
  q = _dot(x, w2_ref[...]) + b2_ref[...]
  query = jax.nn.sigmoid(q)
  query_ref[...] = query
  objl_ref[...] = query * om_ref[...]


def _stage2_body(lsv_ref, cv_ref, cs_ref, c_ref,
                 wa_ref, wb_ref, wc_ref, b1_ref, w2a_ref, w2b_ref,
                 b2a_ref, b2b_ref,
                 newc_ref, msg_ref, maskf_ref):
  lsv = lsv_ref[...]
  cv = cv_ref[...]
  cons = c_ref[...]
  rs = 1.0 / jnp.maximum(cs_ref[...], 1e-9)
  cl = jax.nn.relu(lsv - cv) * rs
  cl1 = jax.nn.relu(cv - lsv) * rs
  pre = (_dot(cons, wa_ref[...]) + _dot(cl, wb_ref[...])
         + _dot(cl1, wc_ref[...]) + b1_ref[...])
  h = jax.nn.relu(_norm(pre))
  newc_ref[...] = _dot(h, w2a_ref[...]) + b2a_ref[...] + 0.5 * cons
  msg_ref[...] = _dot(h, w2b_ref[...]) + b2b_ref[...]
  maskf_ref[...] = (lsv > cv).astype(jnp.float32)


def _stage3_body(v_ref, c2v_ref, gp_ref, objl_ref, om_ref, vs_ref,
                 wa_ref, wb_ref, wc_ref, wd_ref, we_ref, b1_ref,
                 w2_ref, b2_ref, ow1_ref, ob1_ref, ow2_ref, ob2_ref,
                 newv_ref, sig_ref, ov_ref):
  v = v_ref[...]
  om = om_ref[...]
  rs = 1.0 / jnp.maximum(vs_ref[...], 1e-9)
  c2v = c2v_ref[...] * rs
  grad = gp_ref[...] + om
  pre = (_dot(v, wa_ref[...]) + _dot(c2v, wb_ref[...])
         + _dot(objl_ref[...], wc_ref[...]) + _dot(grad, wd_ref[...])
         + om * we_ref[...] + b1_ref[...])
  h2 = jax.nn.relu(_norm(pre))
  newv = _dot(h2, w2_ref[...]) + b2_ref[...] + 0.5 * v
  newv_ref[...] = newv
  h3 = jax.nn.relu(_norm(_dot(newv, ow1_ref[...]) + ob1_ref[...]))
  ov = _dot(h3, ow2_ref[...]) + ob2_ref[...]
  ov_ref[...] = ov
  sig_ref[...] = jax.nn.sigmoid(ov)


def _row_spec(w):
  return pl.BlockSpec((ROW_R, w), lambda i: (i, 0))


def _full_spec(shape):
  return pl.BlockSpec(shape, lambda i: (0, 0))


def _make_stage(body, n, in_widths, full_shapes, out_widths):
  grid = (n // ROW_R,)
  in_specs = ([_row_spec(w) for w in in_widths]
              + [_full_spec(s) for s in full_shapes])
  out_specs = tuple(_row_spec(w) for w in out_widths)
  out_shape = tuple(jax.ShapeDtypeStruct((n, w), jnp.float32)
                    for w in out_widths)
  return pl.pallas_call(body, grid=grid, in_specs=in_specs,
                        out_specs=out_specs, out_shape=out_shape)


def kernel(edge_index, edge_values, const_values, objective_multipliers,
           integer_mask, cu_W1, cu_b1, cu_W2, cu_b2, mq_W1, mq_b1,
           mq_W2, mq_b2, vu_W1, vu_b1, vu_W2, vu_b2,
           out_W1, out_b1, out_W2, out_b2):
  nv = objective_multipliers.shape[0]
  nc = const_values.shape[0]
  ne = edge_values.shape[0]
  ne_pad = ((ne + 8191) // 8192) * 8192

  src = edge_index[0]
  dst = edge_index[1]
  zi = jnp.zeros((ne_pad - ne,), src.dtype)
  zf = jnp.zeros((ne_pad - ne,), jnp.float32)
  src_p = jnp.concatenate([src, zi])
  dst_p = jnp.concatenate([dst, zi])
  val_p = jnp.concatenate([edge_values, zf])

  segsum_c = _make_segsum(nc, ne_pad, FM)
  segsum_v = _make_segsum(nv, ne_pad, FM)
  scalers = _make_scalers(nc, ne_pad)

  cs, vs = scalers(src_p, dst_p, val_p)
  cs = cs.reshape(nc, 1)
  vs = vs.reshape(nv, 1)
  cv = const_values.reshape(nc, 1)
  om = objective_multipliers.reshape(nv, 1)

  b = lambda x: x.reshape(1, -1)
  stage1 = _make_stage(_stage1_body, nv, [FM, 1],
                       [(FM, FM), (1, FM), (FM, FM), (1, FM)], [FM, FM])
  stage2 = _make_stage(_stage2_body, nc, [FM, 1, 1, FM],
                       [(FM, FM)] * 3 + [(1, FM), (FM, FM), (FM, FM),
                                         (1, FM), (1, FM)],
                       [FM, FM, FM])
  stage3 = _make_stage(_stage3_body, nv, [FM, FM, FM, FM, 1, 1],
                       [(FM, FM)] * 4 + [(1, FM), (1, FM), (FM, FM), (1, FM),
                                         (FM, FM), (1, FM), (FM, 1), (1, 1)],
                       [FM, 1, 1])

  cuWa, cuWb, cuWc = cu_W1[:FM], cu_W1[FM:2 * FM], cu_W1[2 * FM:]
  cuW2a, cuW2b = cu_W2[:, :FM], cu_W2[:, FM:]
  cub2a, cub2b = cu_b2[:FM], cu_b2[FM:]
  vuWa, vuWb = vu_W1[:FM], vu_W1[FM:2 * FM]
  vuWc, vuWd = vu_W1[2 * FM:3 * FM], vu_W1[3 * FM:4 * FM]
  vuWe = vu_W1[4 * FM:]

  variables = jnp.ones((nv, FM), jnp.float32)
  constraints = jnp.ones((nc, FM), jnp.float32)

  outs = []
  ov = None
  for _ in range(3):
    query, obj_loss = stage1(variables, om, mq_W1, b(mq_b1), mq_W2, b(mq_b2))
    lsv = segsum_c(src_p, dst_p, val_p, query)
    constraints, msg, maskf = stage2(
        lsv, cv, cs, constraints, cuWa, cuWb, cuWc, b(cu_b1),
        cuW2a, cuW2b, b(cub2a), b(cub2b))
    c2v = segsum_v(dst_p, src_p, val_p, msg)
    gpart = segsum_v(dst_p, src_p, val_p, maskf)
    variables, sig, ov = stage3(
        variables, c2v, gpart, obj_loss, om, vs,
        vuWa, vuWb, vuWc, vuWd, b(vuWe), b(vu_b1), vu_W2, b(vu_b2),
        out_W1, b(out_b1), out_W2, b(out_b2))
    outs.append(sig)

  return (outs[0], outs[1], outs[2], ov)


# speedup vs baseline: 5.7287x; 2.7225x over previous
import functools
import jax
import jax.numpy as jnp
from jax import lax
from jax.experimental import pallas as pl
from jax.experimental.pallas import tpu as pltpu
from jax.experimental.pallas import tpu_sc as plsc

FM = 64
FH = 32
CHUNK = 512
SUB = 128
ROW_R = 2000

_GD = lax.GatherDimensionNumbers(offset_dims=(), collapsed_slice_dims=(0,),
                                 start_index_map=(0,))


def _bcast_lane(vv, j):
  idx = jnp.full((16, 1), j, jnp.int32)
  return lax.gather(vv, idx, _GD, (1,),
                    mode=lax.GatherScatterMode.PROMISE_IN_BOUNDS)


def _make_segsum(n_out, n_edges_pad, fh):
  wrow = 125
  acc_rows = ((n_out + 1 + wrow - 1) // wrow) * wrow
  junk = acc_rows - 1
  tile_edges = n_edges_pad // 16
  nchunk = tile_edges // CHUNK
  zchunks = acc_rows // wrow
  wchunks = n_out // wrow
  mesh = plsc.VectorSubcoreMesh(core_axis_name="c", subcore_axis_name="s")

  def body(gidx, sidx, vals, table, out,
           graw, draw, vraw, gidx2, sloc2, rows, wstage, acc, sem):
    cid = lax.axis_index("c")
    sid = lax.axis_index("s")

    def zb(r, _):
      for k in range(fh // 16):
        wstage[r, pl.ds(k * 16, 16)] = jnp.zeros((16,), jnp.float32)
      return _
    lax.fori_loop(0, wrow, zb, None)
    for j in range((zchunks + 15) // 16):
      c = sid + 16 * j
      @pl.when(c < zchunks)
      def _():
        pltpu.sync_copy(wstage, acc.at[pl.ds(c * wrow, wrow), :])
    plsc.subcore_barrier()

    ebase = sid * tile_edges
    tab_c = table.at[cid]
    out_c = out.at[cid]

    def chunk_body(ci, _):
      off = ebase + ci * CHUNK
      pltpu.sync_copy(gidx.at[pl.ds(off, CHUNK)], graw)
      pltpu.sync_copy(sidx.at[pl.ds(off, CHUNK)], draw)
      pltpu.sync_copy(vals.at[pl.ds(off, CHUNK)], vraw)

      def grp(g, _):
        gv = graw[pl.ds(g * 16, 16)]
        d = draw[pl.ds(g * 16, 16)]
        j = g // 8
        col = (g % 8) * 16
        gidx2[j, pl.ds(col, 16)] = gv
        sloc2[j, pl.ds(col, 16)] = d
        return _
      lax.fori_loop(0, CHUNK // 16, grp, None)

      hs = [pltpu.async_copy(tab_c.at[gidx2.at[j]],
                             rows.at[pl.ds(j * SUB, SUB)], sem)
            for j in range(CHUNK // SUB)]
      for h in hs:
        h.wait()

      def scale(g, _):
        vv = vraw[pl.ds(g * 16, 16)]
        for j16 in range(16):
          e = g * 16 + j16
          b = _bcast_lane(vv, j16)
          for k in range(fh // 16):
            r = rows[e, pl.ds(k * 16, 16)]
            rows[e, pl.ds(k * 16, 16)] = r * b
        return _
      lax.fori_loop(0, CHUNK // 16, scale, None)

      for j in range(CHUNK // SUB):
        pltpu.sync_copy(rows.at[pl.ds(j * SUB, SUB)],
                        acc.at[sloc2.at[j]], add=True)
      return _
    lax.fori_loop(0, nchunk, chunk_body, None)
    plsc.subcore_barrier()

    for j in range((wchunks + 15) // 16):
      c = sid + 16 * j
      @pl.when(c < wchunks)
      def _():
        pltpu.sync_copy(acc.at[pl.ds(c * wrow, wrow), :], wstage)
        pltpu.sync_copy(wstage, out_c.at[pl.ds(c * wrow, wrow), :])

  return pl.kernel(
      body,
      out_type=jax.ShapeDtypeStruct((2, n_out, fh), jnp.float32),
      mesh=mesh,
      compiler_params=pltpu.CompilerParams(use_tc_tiling_on_sc=False),
      scratch_types=[
          pltpu.VMEM((CHUNK,), jnp.int32),
          pltpu.VMEM((CHUNK,), jnp.int32),
          pltpu.VMEM((CHUNK,), jnp.float32),
          pltpu.VMEM((CHUNK // SUB, SUB), jnp.int32),
          pltpu.VMEM((CHUNK // SUB, SUB), jnp.int32),
          pltpu.VMEM((CHUNK, fh), jnp.float32),
          pltpu.VMEM((wrow, fh), jnp.float32),
          pltpu.VMEM_SHARED((acc_rows, fh), jnp.float32),
          pltpu.SemaphoreType.DMA,
      ],
  )


def _make_scalers(n_out, n_edges_pad):
  half = n_out // 2
  acc_n = ((half + 1 + 511) // 512) * 512
  junk = acc_n - 1
  tile_edges = n_edges_pad // 16
  nchunk = tile_edges // CHUNK
  zchunks = acc_n // 512
  wchunks = half // 1000
  mesh = plsc.VectorSubcoreMesh(core_axis_name="c", subcore_axis_name="s")

  def body(src, dst, vals, cs_out, vs_out,
           sraw, draw, vraw, sloc2, dloc2, veffs, veffd, zbuf, wstage,
           accc, accv, sem):
    cid = lax.axis_index("c")
    sid = lax.axis_index("s")
    base = cid * half

    def zb(r, _):
      zbuf[pl.ds(r * 16, 16)] = jnp.zeros((16,), jnp.float32)
      return _
    lax.fori_loop(0, 32, zb, None)
    for j in range((zchunks + 15) // 16):
      c = sid + 16 * j
      @pl.when(c < zchunks)
      def _():
        pltpu.sync_copy(zbuf, accc.at[pl.ds(c * 512, 512)])
        pltpu.sync_copy(zbuf, accv.at[pl.ds(c * 512, 512)])
    plsc.subcore_barrier()

    ebase = sid * tile_edges

    def chunk_body(ci, _):
      off = ebase + ci * CHUNK
      pltpu.sync_copy(src.at[pl.ds(off, CHUNK)], sraw)
      pltpu.sync_copy(dst.at[pl.ds(off, CHUNK)], draw)
      pltpu.sync_copy(vals.at[pl.ds(off, CHUNK)], vraw)

      def grp(g, _):
        s = sraw[pl.ds(g * 16, 16)]
        d = draw[pl.ds(g * 16, 16)]
        v = vraw[pl.ds(g * 16, 16)]
        inh_d = (d >= base) & (d < base + half)
        inh_s = (s >= base) & (s < base + half)
        j = g // 8
        col = (g % 8) * 16
        dloc2[j, pl.ds(col, 16)] = jnp.where(inh_d, d - base, junk)
        sloc2[j, pl.ds(col, 16)] = jnp.where(inh_s, s - base, junk)
        veffd[pl.ds(g * 16, 16)] = jnp.where(inh_d, v, 0.0)
        veffs[pl.ds(g * 16, 16)] = jnp.where(inh_s, v, 0.0)
        return _
      lax.fori_loop(0, CHUNK // 16, grp, None)

      for j in range(CHUNK // SUB):
        pltpu.sync_copy(veffd.at[pl.ds(j * SUB, SUB)],
                        accc.at[dloc2.at[j]], add=True)
        pltpu.sync_copy(veffs.at[pl.ds(j * SUB, SUB)],
                        accv.at[sloc2.at[j]], add=True)
      return _
    lax.fori_loop(0, nchunk, chunk_body, None)
    plsc.subcore_barrier()

    for j in range((wchunks + 15) // 16):
      c = sid + 16 * j
      @pl.when(c < wchunks)
      def _():
        pltpu.sync_copy(accc.at[pl.ds(c * 1000, 1000)], wstage)
        pltpu.sync_copy(wstage, cs_out.at[pl.ds(base + c * 1000, 1000)])
        pltpu.sync_copy(accv.at[pl.ds(c * 1000, 1000)], wstage)
        pltpu.sync_copy(wstage, vs_out.at[pl.ds(base + c * 1000, 1000)])

  return pl.kernel(
      body,
      out_type=(jax.ShapeDtypeStruct((n_out,), jnp.float32),
                jax.ShapeDtypeStruct((n_out,), jnp.float32)),
      mesh=mesh,
      compiler_params=pltpu.CompilerParams(use_tc_tiling_on_sc=False),
      scratch_types=[
          pltpu.VMEM((CHUNK,), jnp.int32),
          pltpu.VMEM((CHUNK,), jnp.int32),
          pltpu.VMEM((CHUNK,), jnp.float32),
          pltpu.VMEM((CHUNK // SUB, SUB), jnp.int32),
          pltpu.VMEM((CHUNK // SUB, SUB), jnp.int32),
          pltpu.VMEM((CHUNK,), jnp.float32),
          pltpu.VMEM((CHUNK,), jnp.float32),
          pltpu.VMEM((512,), jnp.float32),
          pltpu.VMEM((1000,), jnp.float32),
          pltpu.VMEM_SHARED((acc_n,), jnp.float32),
          pltpu.VMEM_SHARED((acc_n,), jnp.float32),
          pltpu.SemaphoreType.DMA,
      ],
  )


def _norm(x):
  m = jnp.mean(x, axis=-1, keepdims=True)
  v = jnp.sum((x - m) * (x - m), axis=-1, keepdims=True) * (1.0 / (FM - 1))
  return x / (jnp.sqrt(v) + 1e-5)


def _dot(a, b):
  return jnp.dot(a, b, preferred_element_type=jnp.float32)


def _split2(x):
  return jnp.stack([x[:, :FH], x[:, FH:]])


def _cat2(ref):
  return jnp.concatenate([ref[0], ref[1]], axis=-1)


def _stage1_body(v_ref, om_ref, w1_ref, b1_ref, w2_ref, b2_ref,
                 query_ref, objl_ref):
  x = jax.nn.relu(_dot(v_ref[...], w1_ref[...]) + b1_ref[...])
  q = _dot(x, w2_ref[...]) + b2_ref[...]
  query = jax.nn.sigmoid(q)
  query_ref[...] = _split2(query)
  objl_ref[...] = query * om_ref[...]


def _stage2_common(lsv, cv_ref, cs_ref, c_ref, wa_ref, wb_ref, wc_ref,
                   b1_ref, w2a_ref, w2b_ref, b2a_ref, b2b_ref,
                   newc_ref, msg_ref, maskf_ref):
  cv = cv_ref[...]
  cons = c_ref[...]
  rs = 1.0 / jnp.maximum(cs_ref[...], 1e-9)
  cl = jax.nn.relu(lsv - cv) * rs
  cl1 = jax.nn.relu(cv - lsv) * rs
  pre = (_dot(cons, wa_ref[...]) + _dot(cl, wb_ref[...])
         + _dot(cl1, wc_ref[...]) + b1_ref[...])
  h = jax.nn.relu(_norm(pre))
  newc_ref[...] = _dot(h, w2a_ref[...]) + b2a_ref[...] + 0.5 * cons
  msg_ref[...] = _split2(_dot(h, w2b_ref[...]) + b2b_ref[...])
  maskf_ref[...] = _split2((lsv > cv).astype(jnp.float32))


def _stage2_body(lsv_ref, *args):
  _stage2_common(_cat2(lsv_ref), *args)


def _stage2b_body(q0_ref, cv_ref, cs_ref, c_ref, *args):
  lsv = cs_ref[...] * q0_ref[...]
  _stage2_common(lsv, cv_ref, cs_ref, c_ref, *args)


def _stage3_body(v_ref, c2v_ref, gp_ref, objl_ref, om_ref, vs_ref,
                 wa_ref, wb_ref, wc_ref, wd_ref, we_ref, b1_ref,
                 w2_ref, b2_ref, ow1_ref, ob1_ref, ow2_ref, ob2_ref,
                 newv_ref, sig_ref, ov_ref):
  v = v_ref[...]
  om = om_ref[...]
  rs = 1.0 / jnp.maximum(vs_ref[...], 1e-9)
  c2v = _cat2(c2v_ref) * rs
  grad = _cat2(gp_ref) + om
  pre = (_dot(v, wa_ref[...]) + _dot(c2v, wb_ref[...])
         + _dot(objl_ref[...], wc_ref[...]) + _dot(grad, wd_ref[...])
         + om * we_ref[...] + b1_ref[...])
  h2 = jax.nn.relu(_norm(pre))
  newv = _dot(h2, w2_ref[...]) + b2_ref[...] + 0.5 * v
  newv_ref[...] = newv
  h3 = jax.nn.relu(_norm(_dot(newv, ow1_ref[...]) + ob1_ref[...]))
  ov = _dot(h3, ow2_ref[...]) + ob2_ref[...]
  ov_ref[...] = ov
  sig_ref[...] = jax.nn.sigmoid(ov)


def _row_spec(w):
  return pl.BlockSpec((ROW_R, w), lambda i: (i, 0))


def _fs_spec():
  return pl.BlockSpec((2, ROW_R, FH), lambda i: (0, i, 0))


def _full_spec(shape):
  return pl.BlockSpec(shape, lambda i: (0,) * len(shape))


def _make_stage(body, n, in_widths, full_shapes, out_widths):
  def spec(w):
    if w == 'fs':
      return _fs_spec()
    if isinstance(w, tuple):
      return _full_spec(w)
    return _row_spec(w)
  def shp(w):
    if w == 'fs':
      return jax.ShapeDtypeStruct((2, n, FH), jnp.float32)
    return jax.ShapeDtypeStruct((n, w), jnp.float32)
  grid = (n // ROW_R,)
  in_specs = ([spec(w) for w in in_widths]
              + [_full_spec(s) for s in full_shapes])
  out_specs = tuple(spec(w) for w in out_widths)
  out_shape = tuple(shp(w) for w in out_widths)
  return pl.pallas_call(body, grid=grid, in_specs=in_specs,
                        out_specs=out_specs, out_shape=out_shape)


def kernel(edge_index, edge_values, const_values, objective_multipliers,
           integer_mask, cu_W1, cu_b1, cu_W2, cu_b2, mq_W1, mq_b1,
           mq_W2, mq_b2, vu_W1, vu_b1, vu_W2, vu_b2,
           out_W1, out_b1, out_W2, out_b2):
  nv = objective_multipliers.shape[0]
  nc = const_values.shape[0]
  ne = edge_values.shape[0]
  ne_pad = ((ne + 8191) // 8192) * 8192

  src = edge_index[0]
  dst = edge_index[1]
  zi = jnp.zeros((ne_pad - ne,), src.dtype)
  zf = jnp.zeros((ne_pad - ne,), jnp.float32)
  src_p = jnp.concatenate([src, zi])
  dst_p = jnp.concatenate([dst, zi])
  val_p = jnp.concatenate([edge_values, zf])

  segsum_c = _make_segsum(nc, ne_pad, FH)
  segsum_v = _make_segsum(nv, ne_pad, FH)
  scalers = _make_scalers(nc, ne_pad)

  cs, vs = scalers(src_p, dst_p, val_p)
  cs = cs.reshape(nc, 1)
  vs = vs.reshape(nv, 1)
  cv = const_values.reshape(nc, 1)
  om = objective_multipliers.reshape(nv, 1)

  b = lambda x: x.reshape(1, -1)
  stage1 = _make_stage(_stage1_body, nv, [FM, 1],
                       [(FM, FM), (1, FM), (FM, FM), (1, FM)], ['fs', FM])
  w2shapes = ([(FM, FM)] * 3 + [(1, FM), (FM, FM), (FM, FM),
                                (1, FM), (1, FM)])
  stage2 = _make_stage(_stage2_body, nc, ['fs', 1, 1, FM], w2shapes,
                       [FM, 'fs', 'fs'])
  stage2b = _make_stage(_stage2b_body, nc, [(1, FM), 1, 1, FM], w2shapes,
                        [FM, 'fs', 'fs'])
  stage3 = _make_stage(_stage3_body, nv, [FM, 'fs', 'fs', FM, 1, 1],
                       [(FM, FM)] * 4 + [(1, FM), (1, FM), (FM, FM), (1, FM),
                                         (FM, FM), (1, FM), (FM, 1), (1, 1)],
                       [FM, 1, 1])

  cuWa, cuWb, cuWc = cu_W1[:FM], cu_W1[FM:2 * FM], cu_W1[2 * FM:]
  cuW2a, cuW2b = cu_W2[:, :FM], cu_W2[:, FM:]
  cub2a, cub2b = cu_b2[:FM], cu_b2[FM:]
  vuWa, vuWb = vu_W1[:FM], vu_W1[FM:2 * FM]
  vuWc, vuWd = vu_W1[2 * FM:3 * FM], vu_W1[3 * FM:4 * FM]
  vuWe = vu_W1[4 * FM:]

  variables = jnp.ones((nv, FM), jnp.float32)
  constraints = jnp.ones((nc, FM), jnp.float32)

  outs = []
  ov = None
  for step in range(3):
    query, obj_loss = stage1(variables, om, mq_W1, b(mq_b1), mq_W2, b(mq_b2))
    w2args = (cv, cs, constraints, cuWa, cuWb, cuWc, b(cu_b1),
              cuW2a, cuW2b, b(cub2a), b(cub2b))
    if step == 0:
      q0 = query[:, 0, :].reshape(1, FM)
      constraints, msg, maskf = stage2b(q0, *w2args)
    else:
      lsv = segsum_c(src_p, dst_p, val_p, query)
      constraints, msg, maskf = stage2(lsv, *w2args)
    c2v = segsum_v(dst_p, src_p, val_p, msg)
    gpart = segsum_v(dst_p, src_p, val_p, maskf)
    variables, sig, ov = stage3(
        variables, c2v, gpart, obj_loss, om, vs,
        vuWa, vuWb, vuWc, vuWd, b(vuWe), b(vu_b1), vu_W2, b(vu_b2),
        out_W1, b(out_b1), out_W2, b(out_b2))
    outs.append(sig)

  return (outs[0], outs[1], outs[2], ov)


# speedup vs baseline: 7.8940x; 1.3780x over previous
import functools
import jax
import jax.numpy as jnp
from jax import lax
from jax.experimental import pallas as pl
from jax.experimental.pallas import tpu as pltpu
from jax.experimental.pallas import tpu_sc as plsc

FM = 64
FH = 32
CHUNK = 256
SUB = 128
ROW_R = 2000

_GD = lax.GatherDimensionNumbers(offset_dims=(), collapsed_slice_dims=(0,),
                                 start_index_map=(0,))


def _bcast_lane(vv, j):
  idx = jnp.full((16, 1), j, jnp.int32)
  return lax.gather(vv, idx, _GD, (1,),
                    mode=lax.GatherScatterMode.PROMISE_IN_BOUNDS)


def _make_segsum(n_out, n_edges_pad, fh):
  wrow = 125
  acc_rows = ((n_out + 1 + wrow - 1) // wrow) * wrow
  junk = acc_rows - 1
  tile_edges = n_edges_pad // 16
  nchunk = tile_edges // CHUNK
  zchunks = acc_rows // wrow
  wchunks = n_out // wrow
  mesh = plsc.VectorSubcoreMesh(core_axis_name="c", subcore_axis_name="s")

  def body(gidx, sidx, vals, table, out,
           graw0, draw0, vraw0, gidx20, sloc20, rows0,
           graw1, draw1, vraw1, gidx21, sloc21, rows1,
           wstage, acc, semi0, semg0, sems0, semi1, semg1, sems1):
    cid = lax.axis_index("c")
    sid = lax.axis_index("s")

    slots = ((graw0, draw0, vraw0, gidx20, sloc20, rows0,
              semi0, semg0, sems0),
             (graw1, draw1, vraw1, gidx21, sloc21, rows1,
              semi1, semg1, sems1))

    def zb(r, _):
      for k in range(fh // 16):
        wstage[r, pl.ds(k * 16, 16)] = jnp.zeros((16,), jnp.float32)
      return _
    lax.fori_loop(0, wrow, zb, None)
    for j in range((zchunks + 15) // 16):
      c = sid + 16 * j
      @pl.when(c < zchunks)
      def _():
        pltpu.sync_copy(wstage, acc.at[pl.ds(c * wrow, wrow), :])
    plsc.subcore_barrier()

    ebase = sid * tile_edges
    tab_c = table.at[cid]
    out_c = out.at[cid]

    def issue_idx(c, s):
      graw, draw, vraw, _, _, _, semi, _, _ = slots[s]
      off = ebase + c * CHUNK
      pltpu.async_copy(gidx.at[pl.ds(off, CHUNK)], graw, semi)
      pltpu.async_copy(sidx.at[pl.ds(off, CHUNK)], draw, semi)
      pltpu.async_copy(vals.at[pl.ds(off, CHUNK)], vraw, semi)

    def wait_idx(s):
      graw, draw, vraw, _, _, _, semi, _, _ = slots[s]
      pltpu.make_async_copy(gidx.at[pl.ds(0, CHUNK)], graw, semi).wait()
      pltpu.make_async_copy(sidx.at[pl.ds(0, CHUNK)], draw, semi).wait()
      pltpu.make_async_copy(vals.at[pl.ds(0, CHUNK)], vraw, semi).wait()

    def build(s):
      graw, draw, _, gidx2, sloc2, _, _, _, _ = slots[s]
      def grp(g, _):
        gv = graw[pl.ds(g * 16, 16)]
        d = draw[pl.ds(g * 16, 16)]
        j = g // 8
        col = (g % 8) * 16
        gidx2[j, pl.ds(col, 16)] = gv
        sloc2[j, pl.ds(col, 16)] = d
        return _
      lax.fori_loop(0, CHUNK // 16, grp, None)

    def fire_gathers(s):
      _, _, _, gidx2, _, rows, _, semg, _ = slots[s]
      return [pltpu.async_copy(tab_c.at[gidx2.at[j]],
                               rows.at[pl.ds(j * SUB, SUB)], semg)
              for j in range(CHUNK // SUB)]

    def scale(s):
      _, _, vraw, _, _, rows, _, _, _ = slots[s]
      def go(g, _):
        vv = vraw[pl.ds(g * 16, 16)]
        for j16 in range(16):
          e = g * 16 + j16
          b = _bcast_lane(vv, j16)
          for k in range(fh // 16):
            r = rows[e, pl.ds(k * 16, 16)]
            rows[e, pl.ds(k * 16, 16)] = r * b
        return _
      lax.fori_loop(0, CHUNK // 16, go, None)

    def fire_scatters(s):
      _, _, _, _, sloc2, rows, _, _, sems = slots[s]
      return [pltpu.async_copy(rows.at[pl.ds(j * SUB, SUB)],
                               acc.at[sloc2.at[j]], sems, add=True)
              for j in range(CHUNK // SUB)]

    issue_idx(0, 0)

    @pl.loop(0, nchunk, step=2)
    def pair(ci):
      issue_idx(ci + 1, 1)
      wait_idx(0)
      build(0)
      g0 = fire_gathers(0)
      wait_idx(1)
      build(1)
      g1 = fire_gathers(1)
      for h in g0:
        h.wait()
      scale(0)
      s0 = fire_scatters(0)
      for h in g1:
        h.wait()
      scale(1)
      s1 = fire_scatters(1)
      @pl.when(ci + 2 < nchunk)
      def _():
        issue_idx(ci + 2, 0)
      for h in s0:
        h.wait()
      for h in s1:
        h.wait()

    plsc.subcore_barrier()

    for j in range((wchunks + 15) // 16):
      c = sid + 16 * j
      @pl.when(c < wchunks)
      def _():
        pltpu.sync_copy(acc.at[pl.ds(c * wrow, wrow), :], wstage)
        pltpu.sync_copy(wstage, out_c.at[pl.ds(c * wrow, wrow), :])

  return pl.kernel(
      body,
      out_type=jax.ShapeDtypeStruct((2, n_out, fh), jnp.float32),
      mesh=mesh,
      compiler_params=pltpu.CompilerParams(use_tc_tiling_on_sc=False),
      scratch_types=(
          [pltpu.VMEM((CHUNK,), jnp.int32),
           pltpu.VMEM((CHUNK,), jnp.int32),
           pltpu.VMEM((CHUNK,), jnp.float32),
           pltpu.VMEM((CHUNK // SUB, SUB), jnp.int32),
           pltpu.VMEM((CHUNK // SUB, SUB), jnp.int32),
           pltpu.VMEM((CHUNK, fh), jnp.float32)] * 2
          + [pltpu.VMEM((wrow, fh), jnp.float32),
             pltpu.VMEM_SHARED((acc_rows, fh), jnp.float32)]
          + [pltpu.SemaphoreType.DMA] * 6
      ),
  )


def _make_scalers(n_out, n_edges_pad):
  half = n_out // 2
  acc_n = ((half + 1 + 511) // 512) * 512
  junk = acc_n - 1
  tile_edges = n_edges_pad // 16
  nchunk = tile_edges // CHUNK
  zchunks = acc_n // 512
  wchunks = half // 1000
  mesh = plsc.VectorSubcoreMesh(core_axis_name="c", subcore_axis_name="s")

  def body(src, dst, vals, cs_out, vs_out,
           sraw, draw, vraw, sloc2, dloc2, veffs, veffd, zbuf, wstage,
           accc, accv, sem):
    cid = lax.axis_index("c")
    sid = lax.axis_index("s")
    base = cid * half

    def zb(r, _):
      zbuf[pl.ds(r * 16, 16)] = jnp.zeros((16,), jnp.float32)
      return _
    lax.fori_loop(0, 32, zb, None)
    for j in range((zchunks + 15) // 16):
      c = sid + 16 * j
      @pl.when(c < zchunks)
      def _():
        pltpu.sync_copy(zbuf, accc.at[pl.ds(c * 512, 512)])
        pltpu.sync_copy(zbuf, accv.at[pl.ds(c * 512, 512)])
    plsc.subcore_barrier()

    ebase = sid * tile_edges

    def chunk_body(ci, _):
      off = ebase + ci * CHUNK
      pltpu.sync_copy(src.at[pl.ds(off, CHUNK)], sraw)
      pltpu.sync_copy(dst.at[pl.ds(off, CHUNK)], draw)
      pltpu.sync_copy(vals.at[pl.ds(off, CHUNK)], vraw)

      def grp(g, _):
        s = sraw[pl.ds(g * 16, 16)]
        d = draw[pl.ds(g * 16, 16)]
        v = vraw[pl.ds(g * 16, 16)]
        inh_d = (d >= base) & (d < base + half)
        inh_s = (s >= base) & (s < base + half)
        j = g // 8
        col = (g % 8) * 16
        dloc2[j, pl.ds(col, 16)] = jnp.where(inh_d, d - base, junk)
        sloc2[j, pl.ds(col, 16)] = jnp.where(inh_s, s - base, junk)
        veffd[pl.ds(g * 16, 16)] = jnp.where(inh_d, v, 0.0)
        veffs[pl.ds(g * 16, 16)] = jnp.where(inh_s, v, 0.0)
        return _
      lax.fori_loop(0, CHUNK // 16, grp, None)

      for j in range(CHUNK // SUB):
        pltpu.sync_copy(veffd.at[pl.ds(j * SUB, SUB)],
                        accc.at[dloc2.at[j]], add=True)
        pltpu.sync_copy(veffs.at[pl.ds(j * SUB, SUB)],
                        accv.at[sloc2.at[j]], add=True)
      return _
    lax.fori_loop(0, nchunk, chunk_body, None)
    plsc.subcore_barrier()

    for j in range((wchunks + 15) // 16):
      c = sid + 16 * j
      @pl.when(c < wchunks)
      def _():
        pltpu.sync_copy(accc.at[pl.ds(c * 1000, 1000)], wstage)
        pltpu.sync_copy(wstage, cs_out.at[pl.ds(base + c * 1000, 1000)])
        pltpu.sync_copy(accv.at[pl.ds(c * 1000, 1000)], wstage)
        pltpu.sync_copy(wstage, vs_out.at[pl.ds(base + c * 1000, 1000)])

  return pl.kernel(
      body,
      out_type=(jax.ShapeDtypeStruct((n_out,), jnp.float32),
                jax.ShapeDtypeStruct((n_out,), jnp.float32)),
      mesh=mesh,
      compiler_params=pltpu.CompilerParams(use_tc_tiling_on_sc=False),
      scratch_types=[
          pltpu.VMEM((CHUNK,), jnp.int32),
          pltpu.VMEM((CHUNK,), jnp.int32),
          pltpu.VMEM((CHUNK,), jnp.float32),
          pltpu.VMEM((CHUNK // SUB, SUB), jnp.int32),
          pltpu.VMEM((CHUNK // SUB, SUB), jnp.int32),
          pltpu.VMEM((CHUNK,), jnp.float32),
          pltpu.VMEM((CHUNK,), jnp.float32),
          pltpu.VMEM((512,), jnp.float32),
          pltpu.VMEM((1000,), jnp.float32),
          pltpu.VMEM_SHARED((acc_n,), jnp.float32),
          pltpu.VMEM_SHARED((acc_n,), jnp.float32),
          pltpu.SemaphoreType.DMA,
      ],
  )


def _norm(x):
  m = jnp.mean(x, axis=-1, keepdims=True)
  v = jnp.sum((x - m) * (x - m), axis=-1, keepdims=True) * (1.0 / (FM - 1))
  return x / (jnp.sqrt(v) + 1e-5)


def _dot(a, b):
  return jnp.dot(a, b, preferred_element_type=jnp.float32)


def _split2(x):
  return jnp.stack([x[:, :FH], x[:, FH:]])


def _cat2(ref):
  return jnp.concatenate([ref[0], ref[1]], axis=-1)


def _stage1_body(v_ref, om_ref, w1_ref, b1_ref, w2_ref, b2_ref,
                 query_ref, objl_ref):
  x = jax.nn.relu(_dot(v_ref[...], w1_ref[...]) + b1_ref[...])
  q = _dot(x, w2_ref[...]) + b2_ref[...]
  query = jax.nn.sigmoid(q)
  query_ref[...] = _split2(query)
  objl_ref[...] = query * om_ref[...]


def _stage2_common(lsv, cv_ref, cs_ref, c_ref, wa_ref, wb_ref, wc_ref,
                   b1_ref, w2a_ref, w2b_ref, b2a_ref, b2b_ref,
                   newc_ref, msg_ref, maskf_ref):
  cv = cv_ref[...]
  cons = c_ref[...]
  rs = 1.0 / jnp.maximum(cs_ref[...], 1e-9)
  cl = jax.nn.relu(lsv - cv) * rs
  cl1 = jax.nn.relu(cv - lsv) * rs
  pre = (_dot(cons, wa_ref[...]) + _dot(cl, wb_ref[...])
         + _dot(cl1, wc_ref[...]) + b1_ref[...])
  h = jax.nn.relu(_norm(pre))
  newc_ref[...] = _dot(h, w2a_ref[...]) + b2a_ref[...] + 0.5 * cons
  msg_ref[...] = _split2(_dot(h, w2b_ref[...]) + b2b_ref[...])
  maskf_ref[...] = _split2((lsv > cv).astype(jnp.float32))


def _stage2_body(lsv_ref, *args):
  _stage2_common(_cat2(lsv_ref), *args)


def _stage2b_body(q0_ref, cv_ref, cs_ref, c_ref, *args):
  lsv = cs_ref[...] * q0_ref[...]
  _stage2_common(lsv, cv_ref, cs_ref, c_ref, *args)


def _stage3_body(v_ref, c2v_ref, gp_ref, objl_ref, om_ref, vs_ref,
                 wa_ref, wb_ref, wc_ref, wd_ref, we_ref, b1_ref,
                 w2_ref, b2_ref, ow1_ref, ob1_ref, ow2_ref, ob2_ref,
                 newv_ref, sig_ref, ov_ref):
  v = v_ref[...]
  om = om_ref[...]
  rs = 1.0 / jnp.maximum(vs_ref[...], 1e-9)
  c2v = _cat2(c2v_ref) * rs
  grad = _cat2(gp_ref) + om
  pre = (_dot(v, wa_ref[...]) + _dot(c2v, wb_ref[...])
         + _dot(objl_ref[...], wc_ref[...]) + _dot(grad, wd_ref[...])
         + om * we_ref[...] + b1_ref[...])
  h2 = jax.nn.relu(_norm(pre))
  newv = _dot(h2, w2_ref[...]) + b2_ref[...] + 0.5 * v
  newv_ref[...] = newv
  h3 = jax.nn.relu(_norm(_dot(newv, ow1_ref[...]) + ob1_ref[...]))
  ov = _dot(h3, ow2_ref[...]) + ob2_ref[...]
  ov_ref[...] = ov
  sig_ref[...] = jax.nn.sigmoid(ov)


def _row_spec(w):
  return pl.BlockSpec((ROW_R, w), lambda i: (i, 0))


def _fs_spec():
  return pl.BlockSpec((2, ROW_R, FH), lambda i: (0, i, 0))


def _full_spec(shape):
  return pl.BlockSpec(shape, lambda i: (0,) * len(shape))


def _make_stage(body, n, in_widths, full_shapes, out_widths):
  def spec(w):
    if w == 'fs':
      return _fs_spec()
    if isinstance(w, tuple):
      return _full_spec(w)
    return _row_spec(w)
  def shp(w):
    if w == 'fs':
      return jax.ShapeDtypeStruct((2, n, FH), jnp.float32)
    return jax.ShapeDtypeStruct((n, w), jnp.float32)
  grid = (n // ROW_R,)
  in_specs = ([spec(w) for w in in_widths]
              + [_full_spec(s) for s in full_shapes])
  out_specs = tuple(spec(w) for w in out_widths)
  out_shape = tuple(shp(w) for w in out_widths)
  return pl.pallas_call(body, grid=grid, in_specs=in_specs,
                        out_specs=out_specs, out_shape=out_shape)


def kernel(edge_index, edge_values, const_values, objective_multipliers,
           integer_mask, cu_W1, cu_b1, cu_W2, cu_b2, mq_W1, mq_b1,
           mq_W2, mq_b2, vu_W1, vu_b1, vu_W2, vu_b2,
           out_W1, out_b1, out_W2, out_b2):
  nv = objective_multipliers.shape[0]
  nc = const_values.shape[0]
  ne = edge_values.shape[0]
  ne_pad = ((ne + 8191) // 8192) * 8192

  src = edge_index[0]
  dst = edge_index[1]
  zi = jnp.zeros((ne_pad - ne,), src.dtype)
  zf = jnp.zeros((ne_pad - ne,), jnp.float32)
  src_p = jnp.concatenate([src, zi])
  dst_p = jnp.concatenate([dst, zi])
  val_p = jnp.concatenate([edge_values, zf])

  segsum_c = _make_segsum(nc, ne_pad, FH)
  segsum_v = _make_segsum(nv, ne_pad, FH)
  scalers = _make_scalers(nc, ne_pad)

  cs, vs = scalers(src_p, dst_p, val_p)
  cs = cs.reshape(nc, 1)
  vs = vs.reshape(nv, 1)
  cv = const_values.reshape(nc, 1)
  om = objective_multipliers.reshape(nv, 1)

  b = lambda x: x.reshape(1, -1)
  stage1 = _make_stage(_stage1_body, nv, [FM, 1],
                       [(FM, FM), (1, FM), (FM, FM), (1, FM)], ['fs', FM])
  w2shapes = ([(FM, FM)] * 3 + [(1, FM), (FM, FM), (FM, FM),
                                (1, FM), (1, FM)])
  stage2 = _make_stage(_stage2_body, nc, ['fs', 1, 1, FM], w2shapes,
                       [FM, 'fs', 'fs'])
  stage2b = _make_stage(_stage2b_body, nc, [(1, FM), 1, 1, FM], w2shapes,
                        [FM, 'fs', 'fs'])
  stage3 = _make_stage(_stage3_body, nv, [FM, 'fs', 'fs', FM, 1, 1],
                       [(FM, FM)] * 4 + [(1, FM), (1, FM), (FM, FM), (1, FM),
                                         (FM, FM), (1, FM), (FM, 1), (1, 1)],
                       [FM, 1, 1])

  cuWa, cuWb, cuWc = cu_W1[:FM], cu_W1[FM:2 * FM], cu_W1[2 * FM:]
  cuW2a, cuW2b = cu_W2[:, :FM], cu_W2[:, FM:]
  cub2a, cub2b = cu_b2[:FM], cu_b2[FM:]
  vuWa, vuWb = vu_W1[:FM], vu_W1[FM:2 * FM]
  vuWc, vuWd = vu_W1[2 * FM:3 * FM], vu_W1[3 * FM:4 * FM]
  vuWe = vu_W1[4 * FM:]

  variables = jnp.ones((nv, FM), jnp.float32)
  constraints = jnp.ones((nc, FM), jnp.float32)

  outs = []
  ov = None
  for step in range(3):
    query, obj_loss = stage1(variables, om, mq_W1, b(mq_b1), mq_W2, b(mq_b2))
    w2args = (cv, cs, constraints, cuWa, cuWb, cuWc, b(cu_b1),
              cuW2a, cuW2b, b(cub2a), b(cub2b))
    if step == 0:
      q0 = query[:, 0, :].reshape(1, FM)
      constraints, msg, maskf = stage2b(q0, *w2args)
    else:
      lsv = segsum_c(src_p, dst_p, val_p, query)
      constraints, msg, maskf = stage2(lsv, *w2args)
    c2v = segsum_v(dst_p, src_p, val_p, msg)
    gpart = segsum_v(dst_p, src_p, val_p, maskf)
    variables, sig, ov = stage3(
        variables, c2v, gpart, obj_loss, om, vs,
        vuWa, vuWb, vuWc, vuWd, b(vuWe), b(vu_b1), vu_W2, b(vu_b2),
        out_W1, b(out_b1), out_W2, b(out_b2))
    outs.append(sig)

  return (outs[0], outs[1], outs[2], ov)


# speedup vs baseline: 9.3950x; 1.1901x over previous
import functools
import jax
import jax.numpy as jnp
from jax import lax
from jax.experimental import pallas as pl
from jax.experimental.pallas import tpu as pltpu
from jax.experimental.pallas import tpu_sc as plsc

FM = 64
FH = 32
CHUNK = 256
SUB = 128
ROW_R = 2000

_GD = lax.GatherDimensionNumbers(offset_dims=(), collapsed_slice_dims=(0,),
                                 start_index_map=(0,))


def _bcast_lane(vv, j):
  idx = jnp.full((16, 1), j, jnp.int32)
  return lax.gather(vv, idx, _GD, (1,),
                    mode=lax.GatherScatterMode.PROMISE_IN_BOUNDS)


def _make_segsum(n_out, n_edges_pad, fh):
  wrow = 125
  acc_rows = ((n_out + 1 + wrow - 1) // wrow) * wrow
  junk = acc_rows - 1
  tile_edges = n_edges_pad // 16
  nchunk = tile_edges // CHUNK
  zchunks = acc_rows // wrow
  wchunks = n_out // wrow
  mesh = plsc.VectorSubcoreMesh(core_axis_name="c", subcore_axis_name="s")

  nsub = CHUNK // SUB

  def body(gidx, sidx, vals, table, out,
           g20, d20, v20, rows0, g21, d21, v21, rows1,
           wstage, acc, semi0, semg0, sems0, semi1, semg1, sems1):
    cid = lax.axis_index("c")
    sid = lax.axis_index("s")

    slots = ((g20, d20, v20, rows0, semi0, semg0, sems0),
             (g21, d21, v21, rows1, semi1, semg1, sems1))

    def zb(r, _):
      for k in range(fh // 16):
        wstage[r, pl.ds(k * 16, 16)] = jnp.zeros((16,), jnp.float32)
      return _
    lax.fori_loop(0, wrow, zb, None)
    for j in range((zchunks + 15) // 16):
      c = sid + 16 * j
      @pl.when(c < zchunks)
      def _():
        pltpu.sync_copy(wstage, acc.at[pl.ds(c * wrow, wrow), :])
    plsc.subcore_barrier()

    ebase = sid * tile_edges
    tab_c = table.at[cid]
    out_c = out.at[cid]

    def issue_idx(c, s):
      g2, d2, v2, _, semi, _, _ = slots[s]
      roff = (ebase + c * CHUNK) // SUB
      pltpu.async_copy(gidx.at[pl.ds(roff, nsub), :], g2, semi)
      pltpu.async_copy(sidx.at[pl.ds(roff, nsub), :], d2, semi)
      pltpu.async_copy(vals.at[pl.ds(roff, nsub), :], v2, semi)

    def wait_idx(s):
      g2, d2, v2, _, semi, _, _ = slots[s]
      pltpu.make_async_copy(gidx.at[pl.ds(0, nsub), :], g2, semi).wait()
      pltpu.make_async_copy(sidx.at[pl.ds(0, nsub), :], d2, semi).wait()
      pltpu.make_async_copy(vals.at[pl.ds(0, nsub), :], v2, semi).wait()

    def fire_gathers(s):
      g2, _, _, rows, _, semg, _ = slots[s]
      return [pltpu.async_copy(tab_c.at[g2.at[j]],
                               rows.at[pl.ds(j * SUB, SUB)], semg)
              for j in range(nsub)]

    def scale(s):
      _, _, v2, rows, _, _, _ = slots[s]
      def go(g, _):
        j2 = g // 8
        col = (g % 8) * 16
        vv = v2[j2, pl.ds(col, 16)]
        for j16 in range(16):
          e = g * 16 + j16
          b = _bcast_lane(vv, j16)
          for k in range(fh // 16):
            r = rows[e, pl.ds(k * 16, 16)]
            rows[e, pl.ds(k * 16, 16)] = r * b
        return _
      lax.fori_loop(0, CHUNK // 16, go, None)

    def fire_scatters(s):
      _, d2, _, rows, _, _, sems = slots[s]
      return [pltpu.async_copy(rows.at[pl.ds(j * SUB, SUB)],
                               acc.at[d2.at[j]], sems, add=True)
              for j in range(nsub)]

    issue_idx(0, 0)

    @pl.loop(0, nchunk, step=2)
    def pair(ci):
      issue_idx(ci + 1, 1)
      wait_idx(0)
      g0 = fire_gathers(0)
      wait_idx(1)
      g1 = fire_gathers(1)
      for h in g0:
        h.wait()
      scale(0)
      s0 = fire_scatters(0)
      for h in g1:
        h.wait()
      scale(1)
      s1 = fire_scatters(1)
      @pl.when(ci + 2 < nchunk)
      def _():
        issue_idx(ci + 2, 0)
      for h in s0:
        h.wait()
      for h in s1:
        h.wait()

    plsc.subcore_barrier()

    for j in range((wchunks + 15) // 16):
      c = sid + 16 * j
      @pl.when(c < wchunks)
      def _():
        pltpu.sync_copy(acc.at[pl.ds(c * wrow, wrow), :], wstage)
        pltpu.sync_copy(wstage, out_c.at[pl.ds(c * wrow, wrow), :])

  return pl.kernel(
      body,
      out_type=jax.ShapeDtypeStruct((2, n_out, fh), jnp.float32),
      mesh=mesh,
      compiler_params=pltpu.CompilerParams(use_tc_tiling_on_sc=False),
      scratch_types=(
          [pltpu.VMEM((CHUNK // SUB, SUB), jnp.int32),
           pltpu.VMEM((CHUNK // SUB, SUB), jnp.int32),
           pltpu.VMEM((CHUNK // SUB, SUB), jnp.float32),
           pltpu.VMEM((CHUNK, fh), jnp.float32)] * 2
          + [pltpu.VMEM((wrow, fh), jnp.float32),
             pltpu.VMEM_SHARED((acc_rows, fh), jnp.float32)]
          + [pltpu.SemaphoreType.DMA] * 6
      ),
  )


def _make_scalers(n_out, n_edges_pad):
  acc_n = ((n_out + 1023) // 1024) * 1024
  tile_edges = n_edges_pad // 16
  nchunk = tile_edges // CHUNK
  nsub = CHUNK // SUB
  zchunks = acc_n // 1024
  wchunks = n_out // 1000
  mesh = plsc.VectorSubcoreMesh(core_axis_name="c", subcore_axis_name="s")

  def body(idxs, vals, sv_out,
           i20, v20, i21, v21, zbuf, acc, semi0, sems0, semi1, sems1):
    cid = lax.axis_index("c")
    sid = lax.axis_index("s")
    slots = ((i20, v20, semi0, sems0), (i21, v21, semi1, sems1))

    def zb(r, _):
      zbuf[pl.ds(r * 16, 16)] = jnp.zeros((16,), jnp.float32)
      return _
    lax.fori_loop(0, 64, zb, None)
    for j in range((zchunks + 15) // 16):
      c = sid + 16 * j
      @pl.when(c < zchunks)
      def _():
        pltpu.sync_copy(zbuf, acc.at[pl.ds(c * 1024, 1024)])
    plsc.subcore_barrier()

    ebase = sid * tile_edges
    idx_c = idxs.at[cid]

    def issue(c, s):
      i2, v2, semi, _ = slots[s]
      roff = (ebase + c * CHUNK) // SUB
      pltpu.async_copy(idx_c.at[pl.ds(roff, nsub), :], i2, semi)
      pltpu.async_copy(vals.at[pl.ds(roff, nsub), :], v2, semi)

    def wait_idx(s):
      i2, v2, semi, _ = slots[s]
      pltpu.make_async_copy(idx_c.at[pl.ds(0, nsub), :], i2, semi).wait()
      pltpu.make_async_copy(vals.at[pl.ds(0, nsub), :], v2, semi).wait()

    def fire_scatters(s):
      i2, v2, _, sems = slots[s]
      return [pltpu.async_copy(v2.at[j], acc.at[i2.at[j]], sems, add=True)
              for j in range(nsub)]

    issue(0, 0)

    @pl.loop(0, nchunk, step=2)
    def pair(ci):
      issue(ci + 1, 1)
      wait_idx(0)
      s0 = fire_scatters(0)
      wait_idx(1)
      s1 = fire_scatters(1)
      for h in s0:
        h.wait()
      @pl.when(ci + 2 < nchunk)
      def _():
        issue(ci + 2, 0)
      for h in s1:
        h.wait()

    plsc.subcore_barrier()

    for j in range((wchunks + 15) // 16):
      c = sid + 16 * j
      @pl.when(c < wchunks)
      def _():
        pltpu.sync_copy(acc.at[pl.ds(c * 1000, 1000)], zbuf.at[pl.ds(0, 1000)])
        pltpu.sync_copy(zbuf.at[pl.ds(0, 1000)],
                        sv_out.at[cid].at[pl.ds(c * 1000, 1000)])

  return pl.kernel(
      body,
      out_type=jax.ShapeDtypeStruct((2, n_out), jnp.float32),
      mesh=mesh,
      compiler_params=pltpu.CompilerParams(use_tc_tiling_on_sc=False),
      scratch_types=(
          [pltpu.VMEM((CHUNK // SUB, SUB), jnp.int32),
           pltpu.VMEM((CHUNK // SUB, SUB), jnp.float32)] * 2
          + [pltpu.VMEM((1024,), jnp.float32),
             pltpu.VMEM_SHARED((acc_n,), jnp.float32)]
          + [pltpu.SemaphoreType.DMA] * 4
      ),
  )


def _norm(x):
  m = jnp.mean(x, axis=-1, keepdims=True)
  v = jnp.sum((x - m) * (x - m), axis=-1, keepdims=True) * (1.0 / (FM - 1))
  return x / (jnp.sqrt(v) + 1e-5)


def _dot(a, b):
  return jnp.dot(a, b, preferred_element_type=jnp.float32)


def _split2(x):
  return jnp.stack([x[:, :FH], x[:, FH:]])


def _cat2(ref):
  return jnp.concatenate([ref[0], ref[1]], axis=-1)


def _stage1_body(v_ref, om_ref, w1_ref, b1_ref, w2_ref, b2_ref,
                 query_ref, objl_ref):
  x = jax.nn.relu(_dot(v_ref[...], w1_ref[...]) + b1_ref[...])
  q = _dot(x, w2_ref[...]) + b2_ref[...]
  query = jax.nn.sigmoid(q)
  query_ref[...] = _split2(query)
  objl_ref[...] = query * om_ref[...]


def _stage2_common(lsv, cv_ref, cs_ref, c_ref, wa_ref, wb_ref, wc_ref,
                   b1_ref, w2a_ref, w2b_ref, b2a_ref, b2b_ref,
                   newc_ref, msg_ref, maskf_ref):
  cv = cv_ref[...]
  cons = c_ref[...]
  rs = 1.0 / jnp.maximum(cs_ref[...], 1e-9)
  cl = jax.nn.relu(lsv - cv) * rs
  cl1 = jax.nn.relu(cv - lsv) * rs
  pre = (_dot(cons, wa_ref[...]) + _dot(cl, wb_ref[...])
         + _dot(cl1, wc_ref[...]) + b1_ref[...])
  h = jax.nn.relu(_norm(pre))
  newc_ref[...] = _dot(h, w2a_ref[...]) + b2a_ref[...] + 0.5 * cons
  msg_ref[...] = _split2(_dot(h, w2b_ref[...]) + b2b_ref[...])
  maskf_ref[...] = _split2((lsv > cv).astype(jnp.float32))


def _stage2_body(lsv_ref, *args):
  _stage2_common(_cat2(lsv_ref), *args)


def _stage2b_body(q0_ref, cv_ref, cs_ref, c_ref, *args):
  lsv = cs_ref[...] * q0_ref[...]
  _stage2_common(lsv, cv_ref, cs_ref, c_ref, *args)


def _stage3_body(v_ref, c2v_ref, gp_ref, objl_ref, om_ref, vs_ref,
                 wa_ref, wb_ref, wc_ref, wd_ref, we_ref, b1_ref,
                 w2_ref, b2_ref, ow1_ref, ob1_ref, ow2_ref, ob2_ref,
                 newv_ref, sig_ref, ov_ref):
  v = v_ref[...]
  om = om_ref[...]
  rs = 1.0 / jnp.maximum(vs_ref[...], 1e-9)
  c2v = _cat2(c2v_ref) * rs
  grad = _cat2(gp_ref) + om
  pre = (_dot(v, wa_ref[...]) + _dot(c2v, wb_ref[...])
         + _dot(objl_ref[...], wc_ref[...]) + _dot(grad, wd_ref[...])
         + om * we_ref[...] + b1_ref[...])
  h2 = jax.nn.relu(_norm(pre))
  newv = _dot(h2, w2_ref[...]) + b2_ref[...] + 0.5 * v
  newv_ref[...] = newv
  h3 = jax.nn.relu(_norm(_dot(newv, ow1_ref[...]) + ob1_ref[...]))
  ov = _dot(h3, ow2_ref[...]) + ob2_ref[...]
  ov_ref[...] = ov
  sig_ref[...] = jax.nn.sigmoid(ov)


def _row_spec(w):
  return pl.BlockSpec((ROW_R, w), lambda i: (i, 0))


def _fs_spec():
  return pl.BlockSpec((2, ROW_R, FH), lambda i: (0, i, 0))


def _full_spec(shape):
  return pl.BlockSpec(shape, lambda i: (0,) * len(shape))


def _make_stage(body, n, in_widths, full_shapes, out_widths):
  def spec(w):
    if w == 'fs':
      return _fs_spec()
    if isinstance(w, tuple):
      return _full_spec(w)
    return _row_spec(w)
  def shp(w):
    if w == 'fs':
      return jax.ShapeDtypeStruct((2, n, FH), jnp.float32)
    return jax.ShapeDtypeStruct((n, w), jnp.float32)
  grid = (n // ROW_R,)
  in_specs = ([spec(w) for w in in_widths]
              + [_full_spec(s) for s in full_shapes])
  out_specs = tuple(spec(w) for w in out_widths)
  out_shape = tuple(shp(w) for w in out_widths)
  return pl.pallas_call(body, grid=grid, in_specs=in_specs,
                        out_specs=out_specs, out_shape=out_shape)


def kernel(edge_index, edge_values, const_values, objective_multipliers,
           integer_mask, cu_W1, cu_b1, cu_W2, cu_b2, mq_W1, mq_b1,
           mq_W2, mq_b2, vu_W1, vu_b1, vu_W2, vu_b2,
           out_W1, out_b1, out_W2, out_b2):
  nv = objective_multipliers.shape[0]
  nc = const_values.shape[0]
  ne = edge_values.shape[0]
  ne_pad = ((ne + 8191) // 8192) * 8192

  src = edge_index[0]
  dst = edge_index[1]
  zi = jnp.zeros((ne_pad - ne,), src.dtype)
  zf = jnp.zeros((ne_pad - ne,), jnp.float32)
  src_p = jnp.concatenate([src, zi]).reshape(-1, SUB)
  dst_p = jnp.concatenate([dst, zi]).reshape(-1, SUB)
  val_p = jnp.concatenate([edge_values, zf]).reshape(-1, SUB)
  idxs = jnp.stack([dst_p, src_p])

  segsum_c = _make_segsum(nc, ne_pad, FH)
  segsum_v = _make_segsum(nv, ne_pad, FH)
  scalers = _make_scalers(nc, ne_pad)

  sv = scalers(idxs, val_p)
  cs = sv[0].reshape(nc, 1)
  vs = sv[1].reshape(nv, 1)
  cv = const_values.reshape(nc, 1)
  om = objective_multipliers.reshape(nv, 1)

  b = lambda x: x.reshape(1, -1)
  stage1 = _make_stage(_stage1_body, nv, [FM, 1],
                       [(FM, FM), (1, FM), (FM, FM), (1, FM)], ['fs', FM])
  w2shapes = ([(FM, FM)] * 3 + [(1, FM), (FM, FM), (FM, FM),
                                (1, FM), (1, FM)])
  stage2 = _make_stage(_stage2_body, nc, ['fs', 1, 1, FM], w2shapes,
                       [FM, 'fs', 'fs'])
  stage2b = _make_stage(_stage2b_body, nc, [(1, FM), 1, 1, FM], w2shapes,
                        [FM, 'fs', 'fs'])
  stage3 = _make_stage(_stage3_body, nv, [FM, 'fs', 'fs', FM, 1, 1],
                       [(FM, FM)] * 4 + [(1, FM), (1, FM), (FM, FM), (1, FM),
                                         (FM, FM), (1, FM), (FM, 1), (1, 1)],
                       [FM, 1, 1])

  cuWa, cuWb, cuWc = cu_W1[:FM], cu_W1[FM:2 * FM], cu_W1[2 * FM:]
  cuW2a, cuW2b = cu_W2[:, :FM], cu_W2[:, FM:]
  cub2a, cub2b = cu_b2[:FM], cu_b2[FM:]
  vuWa, vuWb = vu_W1[:FM], vu_W1[FM:2 * FM]
  vuWc, vuWd = vu_W1[2 * FM:3 * FM], vu_W1[3 * FM:4 * FM]
  vuWe = vu_W1[4 * FM:]

  variables = jnp.ones((nv, FM), jnp.float32)
  constraints = jnp.ones((nc, FM), jnp.float32)

  outs = []
  ov = None
  for step in range(3):
    query, obj_loss = stage1(variables, om, mq_W1, b(mq_b1), mq_W2, b(mq_b2))
    w2args = (cv, cs, constraints, cuWa, cuWb, cuWc, b(cu_b1),
              cuW2a, cuW2b, b(cub2a), b(cub2b))
    if step == 0:
      q0 = query[:, 0, :].reshape(1, FM)
      constraints, msg, maskf = stage2b(q0, *w2args)
    else:
      lsv = segsum_c(src_p, dst_p, val_p, query)
      constraints, msg, maskf = stage2(lsv, *w2args)
    c2v = segsum_v(dst_p, src_p, val_p, msg)
    gpart = segsum_v(dst_p, src_p, val_p, maskf)
    variables, sig, ov = stage3(
        variables, c2v, gpart, obj_loss, om, vs,
        vuWa, vuWb, vuWc, vuWd, b(vuWe), b(vu_b1), vu_W2, b(vu_b2),
        out_W1, b(out_b1), out_W2, b(out_b2))
    outs.append(sig)

  return (outs[0], outs[1], outs[2], ov)


# speedup vs baseline: 9.5313x; 1.0145x over previous
import functools
import jax
import jax.numpy as jnp
from jax import lax
from jax.experimental import pallas as pl
from jax.experimental.pallas import tpu as pltpu
from jax.experimental.pallas import tpu_sc as plsc

FM = 64
FH = 32
CHUNK = 256
SUB = 128
ROW_R = 2000

_GD = lax.GatherDimensionNumbers(offset_dims=(), collapsed_slice_dims=(0,),
                                 start_index_map=(0,))


def _bcast_lane(vv, j):
  idx = jnp.full((16, 1), j, jnp.int32)
  return lax.gather(vv, idx, _GD, (1,),
                    mode=lax.GatherScatterMode.PROMISE_IN_BOUNDS)


def _make_segsum(n_out, n_edges_pad, fh):
  wrow = 125
  acc_rows = ((n_out + 1 + wrow - 1) // wrow) * wrow
  junk = acc_rows - 1
  tile_edges = n_edges_pad // 16
  nchunk = tile_edges // CHUNK
  zchunks = acc_rows // wrow
  wchunks = n_out // wrow
  mesh = plsc.VectorSubcoreMesh(core_axis_name="c", subcore_axis_name="s")

  nsub = CHUNK // SUB

  def body(gidx, sidx, vals, table, out,
           g20, d20, v20, rows0, g21, d21, v21, rows1,
           wstage, acc, semi0, semg0, sems0, semi1, semg1, sems1):
    cid = lax.axis_index("c")
    sid = lax.axis_index("s")

    slots = ((g20, d20, v20, rows0, semi0, semg0, sems0),
             (g21, d21, v21, rows1, semi1, semg1, sems1))

    def zb(r, _):
      for k in range(fh // 16):
        wstage[r, pl.ds(k * 16, 16)] = jnp.zeros((16,), jnp.float32)
      return _
    lax.fori_loop(0, wrow, zb, None)
    for j in range((zchunks + 15) // 16):
      c = sid + 16 * j
      @pl.when(c < zchunks)
      def _():
        pltpu.sync_copy(wstage, acc.at[pl.ds(c * wrow, wrow), :])
    plsc.subcore_barrier()

    ebase = sid * tile_edges
    tab_c = table.at[cid]
    out_c = out.at[cid]

    def issue_idx(c, s):
      g2, d2, v2, _, semi, _, _ = slots[s]
      roff = (ebase + c * CHUNK) // SUB
      pltpu.async_copy(gidx.at[pl.ds(roff, nsub), :], g2, semi)
      pltpu.async_copy(sidx.at[pl.ds(roff, nsub), :], d2, semi)
      pltpu.async_copy(vals.at[pl.ds(roff, nsub), :], v2, semi)

    def wait_idx(s):
      g2, d2, v2, _, semi, _, _ = slots[s]
      pltpu.make_async_copy(gidx.at[pl.ds(0, nsub), :], g2, semi).wait()
      pltpu.make_async_copy(sidx.at[pl.ds(0, nsub), :], d2, semi).wait()
      pltpu.make_async_copy(vals.at[pl.ds(0, nsub), :], v2, semi).wait()

    def fire_gathers(s):
      g2, _, _, rows, _, semg, _ = slots[s]
      return [pltpu.async_copy(tab_c.at[g2.at[j]],
                               rows.at[pl.ds(j * SUB, SUB)], semg)
              for j in range(nsub)]

    def scale(s):
      _, _, v2, rows, _, _, _ = slots[s]
      def go(g, _):
        j2 = g // 8
        col = (g % 8) * 16
        vv = v2[j2, pl.ds(col, 16)]
        for j16 in range(16):
          e = g * 16 + j16
          b = _bcast_lane(vv, j16)
          for k in range(fh // 16):
            r = rows[e, pl.ds(k * 16, 16)]
            rows[e, pl.ds(k * 16, 16)] = r * b
        return _
      lax.fori_loop(0, CHUNK // 16, go, None)

    def fire_scatters(s):
      _, d2, _, rows, _, _, sems = slots[s]
      return [pltpu.async_copy(rows.at[pl.ds(j * SUB, SUB)],
                               acc.at[d2.at[j]], sems, add=True)
              for j in range(nsub)]

    issue_idx(0, 0)

    @pl.loop(0, nchunk, step=2)
    def pair(ci):
      issue_idx(ci + 1, 1)
      wait_idx(0)
      g0 = fire_gathers(0)
      wait_idx(1)
      g1 = fire_gathers(1)
      for h in g0:
        h.wait()
      scale(0)
      s0 = fire_scatters(0)
      for h in g1:
        h.wait()
      scale(1)
      s1 = fire_scatters(1)
      @pl.when(ci + 2 < nchunk)
      def _():
        issue_idx(ci + 2, 0)
      for h in s0:
        h.wait()
      for h in s1:
        h.wait()

    plsc.subcore_barrier()

    for j in range((wchunks + 15) // 16):
      c = sid + 16 * j
      @pl.when(c < wchunks)
      def _():
        pltpu.sync_copy(acc.at[pl.ds(c * wrow, wrow), :], wstage)
        pltpu.sync_copy(wstage, out_c.at[pl.ds(c * wrow, wrow), :])

  return pl.kernel(
      body,
      out_type=jax.ShapeDtypeStruct((2, n_out, fh), jnp.float32),
      mesh=mesh,
      compiler_params=pltpu.CompilerParams(use_tc_tiling_on_sc=False),
      scratch_types=(
          [pltpu.VMEM((CHUNK // SUB, SUB), jnp.int32),
           pltpu.VMEM((CHUNK // SUB, SUB), jnp.int32),
           pltpu.VMEM((CHUNK // SUB, SUB), jnp.float32),
           pltpu.VMEM((CHUNK, fh), jnp.float32)] * 2
          + [pltpu.VMEM((wrow, fh), jnp.float32),
             pltpu.VMEM_SHARED((acc_rows, fh), jnp.float32)]
          + [pltpu.SemaphoreType.DMA] * 6
      ),
  )


def _make_scalers(n_out, n_edges_pad):
  acc_n = ((n_out + 1023) // 1024) * 1024
  tile_edges = n_edges_pad // 16
  nchunk = tile_edges // CHUNK
  nsub = CHUNK // SUB
  zchunks = acc_n // 1024
  wchunks = n_out // 1000
  mesh = plsc.VectorSubcoreMesh(core_axis_name="c", subcore_axis_name="s")

  def body(idxs, vals, sv_out,
           i20, v20, i21, v21, zbuf, acc, semi0, sems0, semi1, sems1):
    cid = lax.axis_index("c")
    sid = lax.axis_index("s")
    slots = ((i20, v20, semi0, sems0), (i21, v21, semi1, sems1))

    def zb(r, _):
      zbuf[pl.ds(r * 16, 16)] = jnp.zeros((16,), jnp.float32)
      return _
    lax.fori_loop(0, 64, zb, None)
    for j in range((zchunks + 15) // 16):
      c = sid + 16 * j
      @pl.when(c < zchunks)
      def _():
        pltpu.sync_copy(zbuf, acc.at[pl.ds(c * 1024, 1024)])
    plsc.subcore_barrier()

    ebase = sid * tile_edges
    idx_c = idxs.at[cid]

    def issue(c, s):
      i2, v2, semi, _ = slots[s]
      roff = (ebase + c * CHUNK) // SUB
      pltpu.async_copy(idx_c.at[pl.ds(roff, nsub), :], i2, semi)
      pltpu.async_copy(vals.at[pl.ds(roff, nsub), :], v2, semi)

    def wait_idx(s):
      i2, v2, semi, _ = slots[s]
      pltpu.make_async_copy(idx_c.at[pl.ds(0, nsub), :], i2, semi).wait()
      pltpu.make_async_copy(vals.at[pl.ds(0, nsub), :], v2, semi).wait()

    def fire_scatters(s):
      i2, v2, _, sems = slots[s]
      return [pltpu.async_copy(v2.at[j], acc.at[i2.at[j]], sems, add=True)
              for j in range(nsub)]

    issue(0, 0)

    @pl.loop(0, nchunk, step=2)
    def pair(ci):
      issue(ci + 1, 1)
      wait_idx(0)
      s0 = fire_scatters(0)
      wait_idx(1)
      s1 = fire_scatters(1)
      for h in s0:
        h.wait()
      @pl.when(ci + 2 < nchunk)
      def _():
        issue(ci + 2, 0)
      for h in s1:
        h.wait()

    plsc.subcore_barrier()

    for j in range((wchunks + 15) // 16):
      c = sid + 16 * j
      @pl.when(c < wchunks)
      def _():
        pltpu.sync_copy(acc.at[pl.ds(c * 1000, 1000)], zbuf.at[pl.ds(0, 1000)])
        pltpu.sync_copy(zbuf.at[pl.ds(0, 1000)],
                        sv_out.at[cid].at[pl.ds(c * 1000, 1000)])

  return pl.kernel(
      body,
      out_type=jax.ShapeDtypeStruct((2, n_out), jnp.float32),
      mesh=mesh,
      compiler_params=pltpu.CompilerParams(use_tc_tiling_on_sc=False),
      scratch_types=(
          [pltpu.VMEM((CHUNK // SUB, SUB), jnp.int32),
           pltpu.VMEM((CHUNK // SUB, SUB), jnp.float32)] * 2
          + [pltpu.VMEM((1024,), jnp.float32),
             pltpu.VMEM_SHARED((acc_n,), jnp.float32)]
          + [pltpu.SemaphoreType.DMA] * 4
      ),
  )


def _norm(x):
  m = jnp.mean(x, axis=-1, keepdims=True)
  v = jnp.sum((x - m) * (x - m), axis=-1, keepdims=True) * (1.0 / (FM - 1))
  return x / (jnp.sqrt(v) + 1e-5)


def _dot(a, b):
  return jnp.dot(a, b, preferred_element_type=jnp.float32)


def _split2(x):
  return jnp.stack([x[:, :FH], x[:, FH:]])


def _cat2(ref):
  return jnp.concatenate([ref[0], ref[1]], axis=-1)


def _stage1_body(v_ref, om_ref, w1_ref, b1_ref, w2_ref, b2_ref,
                 query_ref, objl_ref):
  x = jax.nn.relu(_dot(v_ref[...], w1_ref[...]) + b1_ref[...])
  q = _dot(x, w2_ref[...]) + b2_ref[...]
  query = jax.nn.sigmoid(q)
  query_ref[...] = _split2(query)
  objl_ref[...] = query * om_ref[...]


def _stage2_common(lsv, cv_ref, cs_ref, c_ref, wa_ref, wb_ref, wc_ref,
                   b1_ref, w2a_ref, w2b_ref, b2a_ref, b2b_ref,
                   newc_ref, msg_ref, maskf_ref):
  cv = cv_ref[...]
  cons = c_ref[...]
  rs = 1.0 / jnp.maximum(cs_ref[...], 1e-9)
  cl = jax.nn.relu(lsv - cv) * rs
  cl1 = jax.nn.relu(cv - lsv) * rs
  pre = (_dot(cons, wa_ref[...]) + _dot(cl, wb_ref[...])
         + _dot(cl1, wc_ref[...]) + b1_ref[...])
  h = jax.nn.relu(_norm(pre))
  newc_ref[...] = _dot(h, w2a_ref[...]) + b2a_ref[...] + 0.5 * cons
  msg_ref[...] = _split2(_dot(h, w2b_ref[...]) + b2b_ref[...])
  maskf_ref[...] = _split2((lsv > cv).astype(jnp.float32))


def _stage2_body(lsv_ref, *args):
  _stage2_common(_cat2(lsv_ref), *args)


def _stage2b_body(q0_ref, cv_ref, cs_ref, c_ref, *args):
  lsv = cs_ref[...] * q0_ref[...]
  _stage2_common(lsv, cv_ref, cs_ref, c_ref, *args)


def _stage3_core(v_ref, c2v_ref, gp_ref, objl_ref, om_ref, vs_ref,
                 wa_ref, wb_ref, wc_ref, wd_ref, we_ref, b1_ref,
                 w2_ref, b2_ref, ow1_ref, ob1_ref, ow2_ref, ob2_ref,
                 newv_ref, sig_ref, ov_ref):
  v = v_ref[...]
  om = om_ref[...]
  rs = 1.0 / jnp.maximum(vs_ref[...], 1e-9)
  c2v = _cat2(c2v_ref) * rs
  grad = _cat2(gp_ref) + om
  pre = (_dot(v, wa_ref[...]) + _dot(c2v, wb_ref[...])
         + _dot(objl_ref[...], wc_ref[...]) + _dot(grad, wd_ref[...])
         + om * we_ref[...] + b1_ref[...])
  h2 = jax.nn.relu(_norm(pre))
  newv = _dot(h2, w2_ref[...]) + b2_ref[...] + 0.5 * v
  newv_ref[...] = newv
  h3 = jax.nn.relu(_norm(_dot(newv, ow1_ref[...]) + ob1_ref[...]))
  ov = _dot(h3, ow2_ref[...]) + ob2_ref[...]
  ov_ref[...] = ov
  sig_ref[...] = jax.nn.sigmoid(ov)
  return newv


def _stage3_body(*args):
  _stage3_core(*args)


def _stage31_body(v_ref, c2v_ref, gp_ref, objl_ref, om_ref, vs_ref,
                  wa_ref, wb_ref, wc_ref, wd_ref, we_ref, b1_ref,
                  w2_ref, b2_ref, ow1_ref, ob1_ref, ow2_ref, ob2_ref,
                  mw1_ref, mb1_ref, mw2_ref, mb2_ref,
                  newv_ref, sig_ref, ov_ref, query_ref, objl2_ref):
  newv = _stage3_core(v_ref, c2v_ref, gp_ref, objl_ref, om_ref, vs_ref,
                      wa_ref, wb_ref, wc_ref, wd_ref, we_ref, b1_ref,
                      w2_ref, b2_ref, ow1_ref, ob1_ref, ow2_ref, ob2_ref,
                      newv_ref, sig_ref, ov_ref)
  x = jax.nn.relu(_dot(newv, mw1_ref[...]) + mb1_ref[...])
  q = _dot(x, mw2_ref[...]) + mb2_ref[...]
  query = jax.nn.sigmoid(q)
  query_ref[...] = _split2(query)
  objl2_ref[...] = query * om_ref[...]


def _row_spec(w):
  return pl.BlockSpec((ROW_R, w), lambda i: (i, 0))


def _fs_spec():
  return pl.BlockSpec((2, ROW_R, FH), lambda i: (0, i, 0))


def _full_spec(shape):
  return pl.BlockSpec(shape, lambda i: (0,) * len(shape))


def _make_stage(body, n, in_widths, full_shapes, out_widths):
  def spec(w):
    if w == 'fs':
      return _fs_spec()
    if isinstance(w, tuple):
      return _full_spec(w)
    return _row_spec(w)
  def shp(w):
    if w == 'fs':
      return jax.ShapeDtypeStruct((2, n, FH), jnp.float32)
    return jax.ShapeDtypeStruct((n, w), jnp.float32)
  grid = (n // ROW_R,)
  in_specs = ([spec(w) for w in in_widths]
              + [_full_spec(s) for s in full_shapes])
  out_specs = tuple(spec(w) for w in out_widths)
  out_shape = tuple(shp(w) for w in out_widths)
  return pl.pallas_call(body, grid=grid, in_specs=in_specs,
                        out_specs=out_specs, out_shape=out_shape)


def kernel(edge_index, edge_values, const_values, objective_multipliers,
           integer_mask, cu_W1, cu_b1, cu_W2, cu_b2, mq_W1, mq_b1,
           mq_W2, mq_b2, vu_W1, vu_b1, vu_W2, vu_b2,
           out_W1, out_b1, out_W2, out_b2):
  nv = objective_multipliers.shape[0]
  nc = const_values.shape[0]
  ne = edge_values.shape[0]
  ne_pad = ((ne + 8191) // 8192) * 8192

  src = edge_index[0]
  dst = edge_index[1]
  zi = jnp.zeros((ne_pad - ne,), src.dtype)
  zf = jnp.zeros((ne_pad - ne,), jnp.float32)
  src_p = jnp.concatenate([src, zi]).reshape(-1, SUB)
  dst_p = jnp.concatenate([dst, zi]).reshape(-1, SUB)
  val_p = jnp.concatenate([edge_values, zf]).reshape(-1, SUB)
  idxs = jnp.stack([dst_p, src_p])

  segsum_c = _make_segsum(nc, ne_pad, FH)
  segsum_v = _make_segsum(nv, ne_pad, FH)
  scalers = _make_scalers(nc, ne_pad)

  sv = scalers(idxs, val_p)
  cs = sv[0].reshape(nc, 1)
  vs = sv[1].reshape(nv, 1)
  cv = const_values.reshape(nc, 1)
  om = objective_multipliers.reshape(nv, 1)

  b = lambda x: x.reshape(1, -1)
  stage1 = _make_stage(_stage1_body, nv, [FM, 1],
                       [(FM, FM), (1, FM), (FM, FM), (1, FM)], ['fs', FM])
  w2shapes = ([(FM, FM)] * 3 + [(1, FM), (FM, FM), (FM, FM),
                                (1, FM), (1, FM)])
  stage2 = _make_stage(_stage2_body, nc, ['fs', 1, 1, FM], w2shapes,
                       [FM, 'fs', 'fs'])
  stage2b = _make_stage(_stage2b_body, nc, [(1, FM), 1, 1, FM], w2shapes,
                        [FM, 'fs', 'fs'])
  w3shapes = ([(FM, FM)] * 4 + [(1, FM), (1, FM), (FM, FM), (1, FM),
                                (FM, FM), (1, FM), (FM, 1), (1, 1)])
  stage3 = _make_stage(_stage3_body, nv, [FM, 'fs', 'fs', FM, 1, 1],
                       w3shapes, [FM, 1, 1])
  stage31 = _make_stage(_stage31_body, nv, [FM, 'fs', 'fs', FM, 1, 1],
                        w3shapes + [(FM, FM), (1, FM), (FM, FM), (1, FM)],
                        [FM, 1, 1, 'fs', FM])

  cuWa, cuWb, cuWc = cu_W1[:FM], cu_W1[FM:2 * FM], cu_W1[2 * FM:]
  cuW2a, cuW2b = cu_W2[:, :FM], cu_W2[:, FM:]
  cub2a, cub2b = cu_b2[:FM], cu_b2[FM:]
  vuWa, vuWb = vu_W1[:FM], vu_W1[FM:2 * FM]
  vuWc, vuWd = vu_W1[2 * FM:3 * FM], vu_W1[3 * FM:4 * FM]
  vuWe = vu_W1[4 * FM:]

  variables = jnp.ones((nv, FM), jnp.float32)
  constraints = jnp.ones((nc, FM), jnp.float32)

  outs = []
  ov = None
  query, obj_loss = stage1(variables, om, mq_W1, b(mq_b1), mq_W2, b(mq_b2))
  w3args = (vuWa, vuWb, vuWc, vuWd, b(vuWe), b(vu_b1), vu_W2, b(vu_b2),
            out_W1, b(out_b1), out_W2, b(out_b2))
  for step in range(3):
    w2args = (cv, cs, constraints, cuWa, cuWb, cuWc, b(cu_b1),
              cuW2a, cuW2b, b(cub2a), b(cub2b))
    if step == 0:
      q0 = query[:, 0, :].reshape(1, FM)
      constraints, msg, maskf = stage2b(q0, *w2args)
    else:
      lsv = segsum_c(src_p, dst_p, val_p, query)
      constraints, msg, maskf = stage2(lsv, *w2args)
    c2v = segsum_v(dst_p, src_p, val_p, msg)
    gpart = segsum_v(dst_p, src_p, val_p, maskf)
    if step < 2:
      variables, sig, ov, query, obj_loss = stage31(
          variables, c2v, gpart, obj_loss, om, vs, *w3args,
          mq_W1, b(mq_b1), mq_W2, b(mq_b2))
    else:
      variables, sig, ov = stage3(
          variables, c2v, gpart, obj_loss, om, vs, *w3args)
    outs.append(sig)

  return (outs[0], outs[1], outs[2], ov)


# speedup vs baseline: 9.7425x; 1.0222x over previous
import functools
import jax
import jax.numpy as jnp
from jax import lax
from jax.experimental import pallas as pl
from jax.experimental.pallas import tpu as pltpu
from jax.experimental.pallas import tpu_sc as plsc

FM = 64
FH = 32
CHUNK = 256
SUB = 128
ROW_R = 2000

_GD = lax.GatherDimensionNumbers(offset_dims=(), collapsed_slice_dims=(0,),
                                 start_index_map=(0,))


def _bcast_lane(vv, j):
  idx = jnp.full((16, 1), j, jnp.int32)
  return lax.gather(vv, idx, _GD, (1,),
                    mode=lax.GatherScatterMode.PROMISE_IN_BOUNDS)


def _make_segsum(n_out, n_edges_pad, fh):
  wrow = 125
  acc_rows = ((n_out + 1 + wrow - 1) // wrow) * wrow
  junk = acc_rows - 1
  tile_edges = n_edges_pad // 16
  nchunk = tile_edges // CHUNK
  zchunks = acc_rows // wrow
  wchunks = n_out // wrow
  mesh = plsc.VectorSubcoreMesh(core_axis_name="c", subcore_axis_name="s")

  nsub = CHUNK // SUB

  def body(gidx, sidx, vals, table, out,
           g20, d20, v20, rows0, g21, d21, v21, rows1,
           wstage, acc, semi0, semg0, sems0, semi1, semg1, sems1):
    cid = lax.axis_index("c")
    sid = lax.axis_index("s")

    slots = ((g20, d20, v20, rows0, semi0, semg0, sems0),
             (g21, d21, v21, rows1, semi1, semg1, sems1))

    def zb(r, _):
      for k in range(fh // 16):
        wstage[r, pl.ds(k * 16, 16)] = jnp.zeros((16,), jnp.float32)
      return _
    lax.fori_loop(0, wrow, zb, None)
    for j in range((zchunks + 15) // 16):
      c = sid + 16 * j
      @pl.when(c < zchunks)
      def _():
        pltpu.sync_copy(wstage, acc.at[pl.ds(c * wrow, wrow), :])
    plsc.subcore_barrier()

    ebase = sid * tile_edges
    tab_c = table.at[cid]
    out_c = out.at[cid]

    def issue_idx(c, s):
      g2, d2, v2, _, semi, _, _ = slots[s]
      roff = (ebase + c * CHUNK) // SUB
      pltpu.async_copy(gidx.at[pl.ds(roff, nsub), :], g2, semi)
      pltpu.async_copy(sidx.at[pl.ds(roff, nsub), :], d2, semi)
      pltpu.async_copy(vals.at[pl.ds(roff, nsub), :], v2, semi)

    def wait_idx(s):
      g2, d2, v2, _, semi, _, _ = slots[s]
      pltpu.make_async_copy(gidx.at[pl.ds(0, nsub), :], g2, semi).wait()
      pltpu.make_async_copy(sidx.at[pl.ds(0, nsub), :], d2, semi).wait()
      pltpu.make_async_copy(vals.at[pl.ds(0, nsub), :], v2, semi).wait()

    def fire_gathers(s):
      g2, _, _, rows, _, semg, _ = slots[s]
      return [pltpu.async_copy(tab_c.at[g2.at[j]],
                               rows.at[pl.ds(j * SUB, SUB)], semg)
              for j in range(nsub)]

    def scale(s):
      _, _, v2, rows, _, _, _ = slots[s]
      def go(g, _):
        j2 = g // 8
        col = (g % 8) * 16
        vv = v2[j2, pl.ds(col, 16)]
        for j16 in range(16):
          e = g * 16 + j16
          b = _bcast_lane(vv, j16)
          for k in range(fh // 16):
            r = rows[e, pl.ds(k * 16, 16)]
            rows[e, pl.ds(k * 16, 16)] = r * b
        return _
      lax.fori_loop(0, CHUNK // 16, go, None)

    def fire_scatters(s):
      _, d2, _, rows, _, _, sems = slots[s]
      return [pltpu.async_copy(rows.at[pl.ds(j * SUB, SUB)],
                               acc.at[d2.at[j]], sems, add=True)
              for j in range(nsub)]

    issue_idx(0, 0)
    issue_idx(1, 1)

    @pl.loop(0, nchunk, step=2)
    def pair(ci):
      wait_idx(0)
      g0 = fire_gathers(0)
      wait_idx(1)
      g1 = fire_gathers(1)
      for h in g0:
        h.wait()
      scale(0)
      s0 = fire_scatters(0)
      @pl.when(ci + 2 < nchunk)
      def _():
        issue_idx(ci + 2, 0)
      for h in g1:
        h.wait()
      scale(1)
      s1 = fire_scatters(1)
      @pl.when(ci + 3 < nchunk)
      def _():
        issue_idx(ci + 3, 1)
      for h in s0:
        h.wait()
      for h in s1:
        h.wait()

    plsc.subcore_barrier()

    for j in range((wchunks + 15) // 16):
      c = sid + 16 * j
      @pl.when(c < wchunks)
      def _():
        pltpu.sync_copy(acc.at[pl.ds(c * wrow, wrow), :], wstage)
        pltpu.sync_copy(wstage, out_c.at[pl.ds(c * wrow, wrow), :])

  return pl.kernel(
      body,
      out_type=jax.ShapeDtypeStruct((2, n_out, fh), jnp.float32),
      mesh=mesh,
      compiler_params=pltpu.CompilerParams(use_tc_tiling_on_sc=False),
      scratch_types=(
          [pltpu.VMEM((CHUNK // SUB, SUB), jnp.int32),
           pltpu.VMEM((CHUNK // SUB, SUB), jnp.int32),
           pltpu.VMEM((CHUNK // SUB, SUB), jnp.float32),
           pltpu.VMEM((CHUNK, fh), jnp.float32)] * 2
          + [pltpu.VMEM((wrow, fh), jnp.float32),
             pltpu.VMEM_SHARED((acc_rows, fh), jnp.float32)]
          + [pltpu.SemaphoreType.DMA] * 6
      ),
  )


def _make_scalers(n_out, n_edges_pad):
  acc_n = ((n_out + 1023) // 1024) * 1024
  tile_edges = n_edges_pad // 16
  nchunk = tile_edges // CHUNK
  nsub = CHUNK // SUB
  zchunks = acc_n // 1024
  wchunks = n_out // 1000
  mesh = plsc.VectorSubcoreMesh(core_axis_name="c", subcore_axis_name="s")

  def body(idxs, vals, sv_out,
           i20, v20, i21, v21, zbuf, acc, semi0, sems0, semi1, sems1):
    cid = lax.axis_index("c")
    sid = lax.axis_index("s")
    slots = ((i20, v20, semi0, sems0), (i21, v21, semi1, sems1))

    def zb(r, _):
      zbuf[pl.ds(r * 16, 16)] = jnp.zeros((16,), jnp.float32)
      return _
    lax.fori_loop(0, 64, zb, None)
    for j in range((zchunks + 15) // 16):
      c = sid + 16 * j
      @pl.when(c < zchunks)
      def _():
        pltpu.sync_copy(zbuf, acc.at[pl.ds(c * 1024, 1024)])
    plsc.subcore_barrier()

    ebase = sid * tile_edges
    idx_c = idxs.at[cid]

    def issue(c, s):
      i2, v2, semi, _ = slots[s]
      roff = (ebase + c * CHUNK) // SUB
      pltpu.async_copy(idx_c.at[pl.ds(roff, nsub), :], i2, semi)
      pltpu.async_copy(vals.at[pl.ds(roff, nsub), :], v2, semi)

    def wait_idx(s):
      i2, v2, semi, _ = slots[s]
      pltpu.make_async_copy(idx_c.at[pl.ds(0, nsub), :], i2, semi).wait()
      pltpu.make_async_copy(vals.at[pl.ds(0, nsub), :], v2, semi).wait()

    def fire_scatters(s):
      i2, v2, _, sems = slots[s]
      return [pltpu.async_copy(v2.at[j], acc.at[i2.at[j]], sems, add=True)
              for j in range(nsub)]

    issue(0, 0)
    issue(1, 1)

    @pl.loop(0, nchunk, step=2)
    def pair(ci):
      wait_idx(0)
      s0 = fire_scatters(0)
      wait_idx(1)
      s1 = fire_scatters(1)
      for h in s0:
        h.wait()
      @pl.when(ci + 2 < nchunk)
      def _():
        issue(ci + 2, 0)
      for h in s1:
        h.wait()
      @pl.when(ci + 3 < nchunk)
      def _():
        issue(ci + 3, 1)

    plsc.subcore_barrier()

    for j in range((wchunks + 15) // 16):
      c = sid + 16 * j
      @pl.when(c < wchunks)
      def _():
        pltpu.sync_copy(acc.at[pl.ds(c * 1000, 1000)], zbuf.at[pl.ds(0, 1000)])
        pltpu.sync_copy(zbuf.at[pl.ds(0, 1000)],
                        sv_out.at[cid].at[pl.ds(c * 1000, 1000)])

  return pl.kernel(
      body,
      out_type=jax.ShapeDtypeStruct((2, n_out), jnp.float32),
      mesh=mesh,
      compiler_params=pltpu.CompilerParams(use_tc_tiling_on_sc=False),
      scratch_types=(
          [pltpu.VMEM((CHUNK // SUB, SUB), jnp.int32),
           pltpu.VMEM((CHUNK // SUB, SUB), jnp.float32)] * 2
          + [pltpu.VMEM((1024,), jnp.float32),
             pltpu.VMEM_SHARED((acc_n,), jnp.float32)]
          + [pltpu.SemaphoreType.DMA] * 4
      ),
  )


def _norm(x):
  m = jnp.mean(x, axis=-1, keepdims=True)
  v = jnp.sum((x - m) * (x - m), axis=-1, keepdims=True) * (1.0 / (FM - 1))
  return x / (jnp.sqrt(v) + 1e-5)


def _dot(a, b):
  return jnp.dot(a, b, preferred_element_type=jnp.float32)


def _split2(x):
  return jnp.stack([x[:, :FH], x[:, FH:]])


def _cat2(ref):
  return jnp.concatenate([ref[0], ref[1]], axis=-1)


def _stage1_body(v_ref, om_ref, w1_ref, b1_ref, w2_ref, b2_ref,
                 query_ref, objl_ref):
  x = jax.nn.relu(_dot(v_ref[...], w1_ref[...]) + b1_ref[...])
  q = _dot(x, w2_ref[...]) + b2_ref[...]
  query = jax.nn.sigmoid(q)
  query_ref[...] = _split2(query)
  objl_ref[...] = query * om_ref[...]


def _stage2_common(lsv, cv_ref, cs_ref, c_ref, wa_ref, wb_ref, wc_ref,
                   b1_ref, w2a_ref, w2b_ref, b2a_ref, b2b_ref,
                   newc_ref, msg_ref, maskf_ref):
  cv = cv_ref[...]
  cons = c_ref[...]
  rs = 1.0 / jnp.maximum(cs_ref[...], 1e-9)
  cl = jax.nn.relu(lsv - cv) * rs
  cl1 = jax.nn.relu(cv - lsv) * rs
  pre = (_dot(cons, wa_ref[...]) + _dot(cl, wb_ref[...])
         + _dot(cl1, wc_ref[...]) + b1_ref[...])
  h = jax.nn.relu(_norm(pre))
  newc_ref[...] = _dot(h, w2a_ref[...]) + b2a_ref[...] + 0.5 * cons
  msg_ref[...] = _split2(_dot(h, w2b_ref[...]) + b2b_ref[...])
  maskf_ref[...] = _split2((lsv > cv).astype(jnp.float32))


def _stage2_body(lsv_ref, *args):
  _stage2_common(_cat2(lsv_ref), *args)


def _stage2b_body(q0_ref, cv_ref, cs_ref, c_ref, *args):
  lsv = cs_ref[...] * q0_ref[...]
  _stage2_common(lsv, cv_ref, cs_ref, c_ref, *args)


def _stage3_core(v_ref, c2v_ref, gp_ref, objl_ref, om_ref, vs_ref,
                 wa_ref, wb_ref, wc_ref, wd_ref, we_ref, b1_ref,
                 w2_ref, b2_ref, ow1_ref, ob1_ref, ow2_ref, ob2_ref,
                 newv_ref, sig_ref, ov_ref):
  v = v_ref[...]
  om = om_ref[...]
  rs = 1.0 / jnp.maximum(vs_ref[...], 1e-9)
  c2v = _cat2(c2v_ref) * rs
  grad = _cat2(gp_ref) + om
  pre = (_dot(v, wa_ref[...]) + _dot(c2v, wb_ref[...])
         + _dot(objl_ref[...], wc_ref[...]) + _dot(grad, wd_ref[...])
         + om * we_ref[...] + b1_ref[...])
  h2 = jax.nn.relu(_norm(pre))
  newv = _dot(h2, w2_ref[...]) + b2_ref[...] + 0.5 * v
  newv_ref[...] = newv
  h3 = jax.nn.relu(_norm(_dot(newv, ow1_ref[...]) + ob1_ref[...]))
  ov = _dot(h3, ow2_ref[...]) + ob2_ref[...]
  ov_ref[...] = ov
  sig_ref[...] = jax.nn.sigmoid(ov)
  return newv


def _stage3_body(*args):
  _stage3_core(*args)


def _stage31_body(v_ref, c2v_ref, gp_ref, objl_ref, om_ref, vs_ref,
                  wa_ref, wb_ref, wc_ref, wd_ref, we_ref, b1_ref,
                  w2_ref, b2_ref, ow1_ref, ob1_ref, ow2_ref, ob2_ref,
                  mw1_ref, mb1_ref, mw2_ref, mb2_ref,
                  newv_ref, sig_ref, ov_ref, query_ref, objl2_ref):
  newv = _stage3_core(v_ref, c2v_ref, gp_ref, objl_ref, om_ref, vs_ref,
                      wa_ref, wb_ref, wc_ref, wd_ref, we_ref, b1_ref,
                      w2_ref, b2_ref, ow1_ref, ob1_ref, ow2_ref, ob2_ref,
                      newv_ref, sig_ref, ov_ref)
  x = jax.nn.relu(_dot(newv, mw1_ref[...]) + mb1_ref[...])
  q = _dot(x, mw2_ref[...]) + mb2_ref[...]
  query = jax.nn.sigmoid(q)
  query_ref[...] = _split2(query)
  objl2_ref[...] = query * om_ref[...]


def _row_spec(w):
  return pl.BlockSpec((ROW_R, w), lambda i: (i, 0))


def _fs_spec():
  return pl.BlockSpec((2, ROW_R, FH), lambda i: (0, i, 0))


def _full_spec(shape):
  return pl.BlockSpec(shape, lambda i: (0,) * len(shape))


def _make_stage(body, n, in_widths, full_shapes, out_widths):
  def spec(w):
    if w == 'fs':
      return _fs_spec()
    if isinstance(w, tuple):
      return _full_spec(w)
    return _row_spec(w)
  def shp(w):
    if w == 'fs':
      return jax.ShapeDtypeStruct((2, n, FH), jnp.float32)
    return jax.ShapeDtypeStruct((n, w), jnp.float32)
  grid = (n // ROW_R,)
  in_specs = ([spec(w) for w in in_widths]
              + [_full_spec(s) for s in full_shapes])
  out_specs = tuple(spec(w) for w in out_widths)
  out_shape = tuple(shp(w) for w in out_widths)
  return pl.pallas_call(body, grid=grid, in_specs=in_specs,
                        out_specs=out_specs, out_shape=out_shape)


def kernel(edge_index, edge_values, const_values, objective_multipliers,
           integer_mask, cu_W1, cu_b1, cu_W2, cu_b2, mq_W1, mq_b1,
           mq_W2, mq_b2, vu_W1, vu_b1, vu_W2, vu_b2,
           out_W1, out_b1, out_W2, out_b2):
  nv = objective_multipliers.shape[0]
  nc = const_values.shape[0]
  ne = edge_values.shape[0]
  ne_pad = ((ne + 8191) // 8192) * 8192

  src = edge_index[0]
  dst = edge_index[1]
  zi = jnp.zeros((ne_pad - ne,), src.dtype)
  zf = jnp.zeros((ne_pad - ne,), jnp.float32)
  src_p = jnp.concatenate([src, zi]).reshape(-1, SUB)
  dst_p = jnp.concatenate([dst, zi]).reshape(-1, SUB)
  val_p = jnp.concatenate([edge_values, zf]).reshape(-1, SUB)
  idxs = jnp.stack([dst_p, src_p])

  segsum_c = _make_segsum(nc, ne_pad, FH)
  segsum_v = _make_segsum(nv, ne_pad, FH)
  scalers = _make_scalers(nc, ne_pad)

  sv = scalers(idxs, val_p)
  cs = sv[0].reshape(nc, 1)
  vs = sv[1].reshape(nv, 1)
  cv = const_values.reshape(nc, 1)
  om = objective_multipliers.reshape(nv, 1)

  b = lambda x: x.reshape(1, -1)
  stage1 = _make_stage(_stage1_body, nv, [FM, 1],
                       [(FM, FM), (1, FM), (FM, FM), (1, FM)], ['fs', FM])
  w2shapes = ([(FM, FM)] * 3 + [(1, FM), (FM, FM), (FM, FM),
                                (1, FM), (1, FM)])
  stage2 = _make_stage(_stage2_body, nc, ['fs', 1, 1, FM], w2shapes,
                       [FM, 'fs', 'fs'])
  stage2b = _make_stage(_stage2b_body, nc, [(1, FM), 1, 1, FM], w2shapes,
                        [FM, 'fs', 'fs'])
  w3shapes = ([(FM, FM)] * 4 + [(1, FM), (1, FM), (FM, FM), (1, FM),
                                (FM, FM), (1, FM), (FM, 1), (1, 1)])
  stage3 = _make_stage(_stage3_body, nv, [FM, 'fs', 'fs', FM, 1, 1],
                       w3shapes, [FM, 1, 1])
  stage31 = _make_stage(_stage31_body, nv, [FM, 'fs', 'fs', FM, 1, 1],
                        w3shapes + [(FM, FM), (1, FM), (FM, FM), (1, FM)],
                        [FM, 1, 1, 'fs', FM])

  cuWa, cuWb, cuWc = cu_W1[:FM], cu_W1[FM:2 * FM], cu_W1[2 * FM:]
  cuW2a, cuW2b = cu_W2[:, :FM], cu_W2[:, FM:]
  cub2a, cub2b = cu_b2[:FM], cu_b2[FM:]
  vuWa, vuWb = vu_W1[:FM], vu_W1[FM:2 * FM]
  vuWc, vuWd = vu_W1[2 * FM:3 * FM], vu_W1[3 * FM:4 * FM]
  vuWe = vu_W1[4 * FM:]

  variables = jnp.ones((nv, FM), jnp.float32)
  constraints = jnp.ones((nc, FM), jnp.float32)

  outs = []
  ov = None
  query, obj_loss = stage1(variables, om, mq_W1, b(mq_b1), mq_W2, b(mq_b2))
  w3args = (vuWa, vuWb, vuWc, vuWd, b(vuWe), b(vu_b1), vu_W2, b(vu_b2),
            out_W1, b(out_b1), out_W2, b(out_b2))
  for step in range(3):
    w2args = (cv, cs, constraints, cuWa, cuWb, cuWc, b(cu_b1),
              cuW2a, cuW2b, b(cub2a), b(cub2b))
    if step == 0:
      q0 = query[:, 0, :].reshape(1, FM)
      constraints, msg, maskf = stage2b(q0, *w2args)
    else:
      lsv = segsum_c(src_p, dst_p, val_p, query)
      constraints, msg, maskf = stage2(lsv, *w2args)
    c2v = segsum_v(dst_p, src_p, val_p, msg)
    gpart = segsum_v(dst_p, src_p, val_p, maskf)
    if step < 2:
      variables, sig, ov, query, obj_loss = stage31(
          variables, c2v, gpart, obj_loss, om, vs, *w3args,
          mq_W1, b(mq_b1), mq_W2, b(mq_b2))
    else:
      variables, sig, ov = stage3(
          variables, c2v, gpart, obj_loss, om, vs, *w3args)
    outs.append(sig)

  return (outs[0], outs[1], outs[2], ov)
